# baseline (speedup 1.0000x reference)
import dataclasses
import functools

import jax
import jax.numpy as jnp
from jax import lax
from jax.experimental import pallas as pl
from jax.experimental.pallas import tpu as pltpu
from jax.experimental.pallas import tpu_sc as plsc

F32 = jnp.float32
I32 = jnp.int32

N = 10000
E = 160000
B = 50
H = 256
N1G = 200
K1 = 100
N1 = B * K1
K2 = 50
N2 = B * K2
NP = 10240
NP1 = 5120
NP2 = 2560
EROWS = 1280
EPAD_IDX = 10100
CAP1 = 16 * 10368
CAP2 = 16 * 10496
SENT1 = N1
SENT2 = N2
HIGHEST = jax.lax.Precision.HIGHEST


def _sc_params():
    cp = pltpu.CompilerParams()
    if "needs_layout_passes" in pltpu.CompilerParams.__dataclass_fields__:
        cp = dataclasses.replace(cp, needs_layout_passes=False)
    return cp


_MESH = plsc.VectorSubcoreMesh(core_axis_name="c", subcore_axis_name="s")


def _bitrsqrt(x):
    i = plsc.bitcast(x, I32)
    i = I32(0x5F3759DF) - (i >> 1)
    y = plsc.bitcast(i, F32)
    for _ in range(3):
        y = y * (1.5 - 0.5 * x * y * y)
    return y


def _iota16():
    return lax.iota(I32, 16)


def _sc1(xp, src1d, dst1d):
    @functools.partial(
        pl.kernel,
        out_type=[jax.ShapeDtypeStruct((NP,), F32),
                  jax.ShapeDtypeStruct((NP,), F32)],
        mesh=_MESH, compiler_params=_sc_params(),
        scratch_types=[
            pltpu.VMEM_SHARED((NP,), F32),
            pltpu.VMEM((NP,), F32),
            pltpu.VMEM((NP,), F32),
            pltpu.VMEM((1024,), I32),
            pltpu.VMEM((1024,), I32),
            pltpu.VMEM((1024,), F32),
        ] + [pltpu.VMEM((128,), I32) for _ in range(8)] + [
            pltpu.VMEM((128,), F32),
            pltpu.VMEM((640,), F32),
            pltpu.SemaphoreType.DMA,
        ],
    )
    def k(x_hbm, src_hbm, dst_hbm, dis_hbm, tacc_hbm,
          acc_sh, xl, dl, srcb, dstb, valb,
          d0, d1, d2, d3, d4, d5, d6, d7, onesr, zb, sem):
        cid = lax.axis_index("c")
        sid = lax.axis_index("s")
        drows = [d0, d1, d2, d3, d4, d5, d6, d7]

        @pl.when(cid == 0)
        def _():
            @pl.loop(0, 40)
            def _(i):
                zb[pl.ds(i * 16, 16)] = jnp.zeros((16,), F32)
            pltpu.sync_copy(zb, acc_sh.at[pl.ds(sid * 640, 640)])
            pltpu.sync_copy(x_hbm, xl)

            @pl.loop(0, 8)
            def _(i):
                onesr[pl.ds(i * 16, 16)] = jnp.full((16,), 1.0, F32)
            plsc.subcore_barrier()

            @pl.loop(0, 10)
            def _(w):
                base = (sid * 10 + w) * 1024
                pltpu.sync_copy(dst_hbm.at[pl.ds(base, 1024)], dstb)
                for j in range(8):
                    for c in range(8):
                        drows[j][pl.ds(c * 16, 16)] = (
                            dstb[pl.ds(j * 128 + c * 16, 16)])
                cps = [pltpu.async_copy(onesr, acc_sh.at[drows[j]], sem,
                                        add=True) for j in range(8)]
                for cp in cps:
                    cp.wait()
            plsc.subcore_barrier()

            pltpu.sync_copy(acc_sh, dl)

            @pl.loop(0, NP // 16)
            def _(i):
                d16 = dl[pl.ds(i * 16, 16)]
                dl[pl.ds(i * 16, 16)] = _bitrsqrt(d16 + 1.0)
            pltpu.sync_copy(dl.at[pl.ds(sid * 640, 640)],
                            dis_hbm.at[pl.ds(sid * 640, 640)])

            @pl.loop(0, NP // 16)
            def _(i):
                xl[pl.ds(i * 16, 16)] = (xl[pl.ds(i * 16, 16)]
                                         * dl[pl.ds(i * 16, 16)])
            plsc.subcore_barrier()
            pltpu.sync_copy(zb, acc_sh.at[pl.ds(sid * 640, 640)])
            plsc.subcore_barrier()

            @pl.loop(0, 10)
            def _(w):
                base = (sid * 10 + w) * 1024
                pltpu.sync_copy(src_hbm.at[pl.ds(base, 1024)], srcb)
                pltpu.sync_copy(dst_hbm.at[pl.ds(base, 1024)], dstb)
                for j in range(8):
                    for c in range(8):
                        o = j * 128 + c * 16
                        s16 = srcb[pl.ds(o, 16)]
                        valb[pl.ds(o, 16)] = plsc.load_gather(xl, [s16])
                        drows[j][pl.ds(c * 16, 16)] = dstb[pl.ds(o, 16)]
                cps = [pltpu.async_copy(valb.at[pl.ds(j * 128, 128)],
                                        acc_sh.at[drows[j]], sem, add=True)
                       for j in range(8)]
                for cp in cps:
                    cp.wait()
            plsc.subcore_barrier()
            pltpu.sync_copy(acc_sh.at[pl.ds(sid * 640, 640)],
                            tacc_hbm.at[pl.ds(sid * 640, 640)])

    return k(xp, src1d, dst1d)


def _scs1(ss1, dis, src1d, dst1d):
    @functools.partial(
        pl.kernel,
        out_type=jax.ShapeDtypeStruct((NP,), F32),
        mesh=_MESH, compiler_params=_sc_params(),
        scratch_types=[
            pltpu.VMEM_SHARED((NP,), F32),
            pltpu.VMEM((NP,), F32),
            pltpu.VMEM((640,), F32),
            pltpu.VMEM((640,), F32),
            pltpu.VMEM((1024,), I32),
            pltpu.VMEM((1024,), I32),
            pltpu.VMEM((1024,), F32),
        ] + [pltpu.VMEM((128,), I32) for _ in range(8)] + [
            pltpu.VMEM((640,), F32),
            pltpu.SemaphoreType.DMA,
        ],
    )
    def k(ss1_hbm, dis_hbm, src_hbm, dst_hbm, sc_hbm,
          acc_sh, sl, disl, sal, srcb, dstb, valb,
          d0, d1, d2, d3, d4, d5, d6, d7, zb, sem):
        cid = lax.axis_index("c")
        sid = lax.axis_index("s")
        drows = [d0, d1, d2, d3, d4, d5, d6, d7]

        @pl.when(cid == 0)
        def _():
            @pl.loop(0, 40)
            def _(i):
                zb[pl.ds(i * 16, 16)] = jnp.zeros((16,), F32)
            pltpu.sync_copy(zb, acc_sh.at[pl.ds(sid * 640, 640)])
            pltpu.sync_copy(ss1_hbm, sl)
            plsc.subcore_barrier()

            @pl.loop(0, 10)
            def _(w):
                base = (sid * 10 + w) * 1024
                pltpu.sync_copy(src_hbm.at[pl.ds(base, 1024)], srcb)
                pltpu.sync_copy(dst_hbm.at[pl.ds(base, 1024)], dstb)
                for j in range(8):
                    for c in range(8):
                        o = j * 128 + c * 16
                        s16 = srcb[pl.ds(o, 16)]
                        valb[pl.ds(o, 16)] = plsc.load_gather(sl, [s16])
                        drows[j][pl.ds(c * 16, 16)] = dstb[pl.ds(o, 16)]
                cps = [pltpu.async_copy(valb.at[pl.ds(j * 128, 128)],
                                        acc_sh.at[drows[j]], sem, add=True)
                       for j in range(8)]
                for cp in cps:
                    cp.wait()
            plsc.subcore_barrier()

            base = sid * 640
            pltpu.sync_copy(dis_hbm.at[pl.ds(base, 640)], disl)
            pltpu.sync_copy(acc_sh.at[pl.ds(base, 640)], sal)

            @pl.loop(0, 40)
            def _(i):
                s = pl.ds(i * 16, 16)
                sg = pl.ds(base + i * 16, 16)
                sal[s] = disl[s] * (sal[s] + sl[sg])
            pltpu.sync_copy(sal, sc_hbm.at[pl.ds(base, 640)])

    return k(ss1, dis, src1d, dst1d)


def _scp1(map1, src1d, dst1d, t, score):
    @functools.partial(
        pl.kernel,
        out_type=[jax.ShapeDtypeStruct((CAP1,), I32),
                  jax.ShapeDtypeStruct((CAP1,), I32),
                  jax.ShapeDtypeStruct((8,), I32),
                  jax.ShapeDtypeStruct((NP1,), F32),
                  jax.ShapeDtypeStruct((NP1,), F32),
                  jax.ShapeDtypeStruct((NP1,), F32)],
        mesh=_MESH, compiler_params=_sc_params(),
        scratch_types=[
            pltpu.VMEM_SHARED((NP1,), I32),
            pltpu.VMEM_SHARED((NP1,), F32),
            pltpu.VMEM_SHARED((128,), I32),
            pltpu.VMEM((NP,), I32),
            pltpu.VMEM((10368,), I32),
            pltpu.VMEM((10368,), I32),
            pltpu.VMEM((1024,), I32),
            pltpu.VMEM((1024,), I32),
            pltpu.VMEM((16,), I32),
            pltpu.VMEM((16,), I32),
            pltpu.VMEM((16,), F32),
            pltpu.VMEM((128,), I32),
            pltpu.VMEM((320,), F32),
            pltpu.VMEM((320,), F32),
            pltpu.VMEM((320,), I32),
            pltpu.VMEM((320,), F32),
            pltpu.VMEM((320,), I32),
            pltpu.SemaphoreType.DMA,
        ],
    )
    def k(map_hbm, src_hbm, dst_hbm, t_hbm, sc_hbm,
          nsc_hbm, ndc_hbm, rc_hbm, dis2_hbm, tv_hbm, sv_hbm,
          perm_sh, deg_sh, cnt_sh, mapl, cbs, cbd, srcb, dstb,
          idxb, ivb, onesb, cntl, fb1, fb2, pb, zbf, zbi, sem):
        cid = lax.axis_index("c")
        sid = lax.axis_index("s")

        @pl.when(cid == 0)
        def _():
            @pl.loop(0, 20)
            def _(i):
                zbf[pl.ds(i * 16, 16)] = jnp.zeros((16,), F32)
                zbi[pl.ds(i * 16, 16)] = jnp.zeros((16,), I32)
            pltpu.sync_copy(zbi, perm_sh.at[pl.ds(sid * 320, 320)])
            pltpu.sync_copy(zbf, deg_sh.at[pl.ds(sid * 320, 320)])
            pltpu.sync_copy(map_hbm, mapl)
            onesb[...] = jnp.full((16,), 1.0, F32)
            plsc.subcore_barrier()

            @pl.loop(0, 40)
            def _(c):
                base = sid * 640 + c * 16
                m16 = mapl[pl.ds(base, 16)]
                keep = m16 < N1
                idxb[...] = jnp.where(keep, m16, N1)
                ivb[...] = jnp.full((16,), base, I32) + _iota16()
                pltpu.sync_copy(ivb, perm_sh.at[idxb])

            def row_body(w, cur):
                base = (sid * 80 + w * 8) * 128
                pltpu.sync_copy(src_hbm.at[pl.ds(base, 1024)], srcb)
                pltpu.sync_copy(dst_hbm.at[pl.ds(base, 1024)], dstb)
                for c in range(64):
                    s16 = srcb[pl.ds(c * 16, 16)]
                    d16 = dstb[pl.ds(c * 16, 16)]
                    ns = plsc.load_gather(mapl, [s16])
                    nd = plsc.load_gather(mapl, [d16])
                    ok = (ns < N1) & (nd < N1) & (s16 < N)
                    plsc.store_compressed(cbs.at[pl.ds(cur, 16)], ns, mask=ok)
                    plsc.store_compressed(cbd.at[pl.ds(cur, 16)], nd, mask=ok)
                    cur = cur + plsc.all_reduce_population_count(ok)[0]
                return cur

            cur = lax.fori_loop(0, 10, row_body, 0)
            for j in range(8):
                cbs[pl.ds(cur + j * 16, 16)] = jnp.full((16,), SENT1, I32)
                cbd[pl.ds(cur + j * 16, 16)] = jnp.full((16,), SENT1, I32)
            myrows = (cur + 127) // 128
            ivb[...] = jnp.full((16,), myrows, I32)
            pltpu.sync_copy(ivb.at[pl.ds(0, 8)], cnt_sh.at[pl.ds(sid * 8, 8)])

            def deg_body(kk, _):
                idxb[...] = cbd[pl.ds(kk * 16, 16)]
                pltpu.sync_copy(onesb, deg_sh.at[idxb], add=True)
                return 0
            lax.fori_loop(0, myrows * 8, deg_body, 0)
            plsc.subcore_barrier()

            pltpu.sync_copy(cnt_sh, cntl)
            cnts = plsc.load_gather(cntl, [_iota16() * 8])
            lanes = _iota16()
            rowoff = jnp.sum(jnp.where(lanes < sid, cnts, 0))
            total = jnp.sum(cnts)

            def out_body(r, _):
                pltpu.sync_copy(cbs.at[pl.ds(r * 128, 128)],
                                nsc_hbm.at[pl.ds((rowoff + r) * 128, 128)])
                pltpu.sync_copy(cbd.at[pl.ds(r * 128, 128)],
                                ndc_hbm.at[pl.ds((rowoff + r) * 128, 128)])
                return 0
            lax.fori_loop(0, myrows, out_body, 0)

            @pl.when(sid == 0)
            def _():
                ivb[...] = jnp.full((16,), total, I32)
                pltpu.sync_copy(ivb.at[pl.ds(0, 8)], rc_hbm)
            plsc.subcore_barrier()

            nbase = sid * 320
            pltpu.sync_copy(deg_sh.at[pl.ds(nbase, 320)], fb1)

            @pl.loop(0, 20)
            def _(i):
                s = pl.ds(i * 16, 16)
                fb1[s] = _bitrsqrt(fb1[s] + 1.0)
            pltpu.sync_copy(fb1, dis2_hbm.at[pl.ds(nbase, 320)])

            pltpu.sync_copy(perm_sh.at[pl.ds(nbase, 320)], pb)
            pltpu.async_copy(t_hbm.at[pb], fb2, sem).wait()
            pltpu.sync_copy(fb2, tv_hbm.at[pl.ds(nbase, 320)])
            pltpu.async_copy(sc_hbm.at[pb], fb2, sem).wait()
            pltpu.sync_copy(fb2, sv_hbm.at[pl.ds(nbase, 320)])

    return k(map1, src1d, dst1d, t, score)


def _scw(hhflat, ns1d, nd1d, rcnt, zflat, npx):
    nflat = npx * 16

    @functools.partial(
        pl.kernel,
        out_type=jax.ShapeDtypeStruct((32 * nflat,), F32),
        mesh=_MESH, compiler_params=_sc_params(),
        scratch_types=[
            pltpu.VMEM((nflat,), F32),
            pltpu.VMEM((128,), I32),
            pltpu.VMEM((128,), I32),
            pltpu.VMEM((2048,), I32),
            pltpu.VMEM((2048,), F32),
            pltpu.VMEM((16,), I32),
            pltpu.SemaphoreType.DMA,
        ],
    )
    def k(hh_hbm, ns_hbm, nd_hbm, rc_hbm, z_hbm, out_hbm,
          accl, srow, drow, gib, ebuf, cntb, sem):
        cid = lax.axis_index("c")
        sid = lax.axis_index("s")
        wid = cid * 16 + sid

        pltpu.sync_copy(rc_hbm, cntb.at[pl.ds(0, 8)])
        r2 = cntb[pl.ds(0, 16)][0]
        pltpu.sync_copy(z_hbm, accl)

        hr = (r2 + 1) // 2
        lo = cid * hr
        hi = jnp.minimum(lo + hr, r2)
        iot = _iota16()
        colbase = sid * 16

        def row_body(r, _):
            pltpu.sync_copy(ns_hbm.at[pl.ds(r * 128, 128)], srow)
            pltpu.sync_copy(nd_hbm.at[pl.ds(r * 128, 128)], drow)

            for kk in range(16):
                for c in range(8):
                    s = pl.ds(c * 16, 16)
                    gib[pl.ds(kk * 128 + c * 16, 16)] = (
                        srow[s] * H + (colbase + kk))
            cps = [pltpu.async_copy(hh_hbm.at[gib.at[pl.ds(kk * 128, 128)]],
                                    ebuf.at[pl.ds(kk * 128, 128)], sem)
                   for kk in range(16)]
            for cp in cps:
                cp.wait()

            def e_body(e, _):
                ef = jnp.full((16,), e, I32)
                tb = plsc.load_gather(drow, [ef]) * 16 + iot
                v = plsc.load_gather(ebuf, [iot * 128 + e])
                plsc.addupdate_scatter(accl, [tb], v)
                return 0
            lax.fori_loop(0, 128, e_body, 0, unroll=4)
            return 0

        lax.fori_loop(lo, hi, row_body, 0)
        pltpu.sync_copy(accl, out_hbm.at[pl.ds(wid * nflat, nflat)])

    return k(hhflat, ns1d, nd1d, rcnt, zflat)


def _scs2(ss2, dis2, ns1d, nd1d, rcnt):
    @functools.partial(
        pl.kernel,
        out_type=jax.ShapeDtypeStruct((NP1,), F32),
        mesh=_MESH, compiler_params=_sc_params(),
        scratch_types=[
            pltpu.VMEM_SHARED((NP1,), F32),
            pltpu.VMEM((NP1,), F32),
            pltpu.VMEM((320,), F32),
            pltpu.VMEM((320,), F32),
            pltpu.VMEM((128,), I32),
            pltpu.VMEM((128,), I32),
            pltpu.VMEM((128,), F32),
            pltpu.VMEM((16,), I32),
            pltpu.VMEM((320,), F32),
            pltpu.SemaphoreType.DMA,
        ],
    )
    def k(ss_hbm, dis_hbm, ns_hbm, nd_hbm, rc_hbm, sc_hbm,
          acc_sh, sl, disl, sal, srow, drow, vrow, cntb, zb, sem):
        cid = lax.axis_index("c")
        sid = lax.axis_index("s")

        @pl.when(cid == 0)
        def _():
            @pl.loop(0, 20)
            def _(i):
                zb[pl.ds(i * 16, 16)] = jnp.zeros((16,), F32)
            pltpu.sync_copy(zb, acc_sh.at[pl.ds(sid * 320, 320)])
            pltpu.sync_copy(ss_hbm, sl)
            pltpu.sync_copy(rc_hbm, cntb.at[pl.ds(0, 8)])
            r2 = cntb[pl.ds(0, 16)][0]
            plsc.subcore_barrier()

            tr = (r2 + 15) // 16
            lo = sid * tr
            hi = jnp.minimum(lo + tr, r2)

            def row_body(r, _):
                pltpu.sync_copy(ns_hbm.at[pl.ds(r * 128, 128)], srow)
                pltpu.sync_copy(nd_hbm.at[pl.ds(r * 128, 128)], drow)
                for c in range(8):
                    s = pl.ds(c * 16, 16)
                    vrow[s] = plsc.load_gather(sl, [srow[s]])
                pltpu.sync_copy(vrow, acc_sh.at[drow], add=True)
                return 0
            lax.fori_loop(lo, hi, row_body, 0)
            plsc.subcore_barrier()

            base = sid * 320
            pltpu.sync_copy(dis_hbm.at[pl.ds(base, 320)], disl)
            pltpu.sync_copy(acc_sh.at[pl.ds(base, 320)], sal)

            @pl.loop(0, 20)
            def _(i):
                s = pl.ds(i * 16, 16)
                sg = pl.ds(base + i * 16, 16)
                sal[s] = disl[s] * (sal[s] + sl[sg])
            pltpu.sync_copy(sal, sc_hbm.at[pl.ds(base, 320)])

    return k(ss2, dis2, ns1d, nd1d, rcnt)


def _scp2(map2, ns1d, nd1d, rcnt, h2, score2):
    @functools.partial(
        pl.kernel,
        out_type=[jax.ShapeDtypeStruct((CAP2,), I32),
                  jax.ShapeDtypeStruct((CAP2,), I32),
                  jax.ShapeDtypeStruct((8,), I32),
                  jax.ShapeDtypeStruct((NP2,), F32),
                  jax.ShapeDtypeStruct((NP2, H), F32),
                  jax.ShapeDtypeStruct((NP2,), F32)],
        mesh=_MESH, compiler_params=_sc_params(),
        scratch_types=[
            pltpu.VMEM_SHARED((NP2,), I32),
            pltpu.VMEM_SHARED((NP2,), F32),
            pltpu.VMEM_SHARED((128,), I32),
            pltpu.VMEM((NP1,), I32),
            pltpu.VMEM((10496,), I32),
            pltpu.VMEM((10496,), I32),
            pltpu.VMEM((1024,), I32),
            pltpu.VMEM((1024,), I32),
            pltpu.VMEM((16,), I32),
            pltpu.VMEM((16,), I32),
            pltpu.VMEM((16,), F32),
            pltpu.VMEM((128,), I32),
            pltpu.VMEM((160,), F32),
            pltpu.VMEM((160,), F32),
            pltpu.VMEM((160,), I32),
            pltpu.VMEM((160, H), F32),
            pltpu.VMEM((160,), F32),
            pltpu.VMEM((160,), I32),
            pltpu.SemaphoreType.DMA,
        ],
    )
    def k(map_hbm, ns_hbm, nd_hbm, rc_hbm, h2_hbm, sc_hbm,
          nsc_hbm, ndc_hbm, rc2_hbm, dis3_hbm, hsel_hbm, sv_hbm,
          perm_sh, deg_sh, cnt_sh, mapl, cbs, cbd, srcb, dstb,
          idxb, ivb, onesb, cntl, fb1, fb2, pb, rowb, zbf, zbi, sem):
        cid = lax.axis_index("c")
        sid = lax.axis_index("s")

        @pl.when(cid == 0)
        def _():
            @pl.loop(0, 10)
            def _(i):
                zbf[pl.ds(i * 16, 16)] = jnp.zeros((16,), F32)
                zbi[pl.ds(i * 16, 16)] = jnp.zeros((16,), I32)
            pltpu.sync_copy(zbi, perm_sh.at[pl.ds(sid * 160, 160)])
            pltpu.sync_copy(zbf, deg_sh.at[pl.ds(sid * 160, 160)])
            pltpu.sync_copy(map_hbm, mapl)
            pltpu.sync_copy(rc_hbm, cntl.at[pl.ds(0, 8)])
            r2in = cntl[pl.ds(0, 16)][0]
            onesb[...] = jnp.full((16,), 1.0, F32)
            plsc.subcore_barrier()

            @pl.loop(0, 20)
            def _(c):
                base = sid * 320 + c * 16
                m16 = mapl[pl.ds(base, 16)]
                keep = m16 < N2
                idxb[...] = jnp.where(keep, m16, N2)
                ivb[...] = jnp.full((16,), base, I32) + _iota16()
                pltpu.sync_copy(ivb, perm_sh.at[idxb])

            tr = (r2in + 15) // 16
            lo = sid * tr
            hi = jnp.minimum(lo + tr, r2in)

            def batch_body(w, cur):
                base = (lo + w * 8) * 128
                pltpu.sync_copy(ns_hbm.at[pl.ds(base, 1024)], srcb)
                pltpu.sync_copy(nd_hbm.at[pl.ds(base, 1024)], dstb)
                for c in range(64):
                    s16 = srcb[pl.ds(c * 16, 16)]
                    d16 = dstb[pl.ds(c * 16, 16)]
                    ns = plsc.load_gather(mapl, [s16])
                    nd = plsc.load_gather(mapl, [d16])
                    ok = (ns < N2) & (nd < N2)
                    plsc.store_compressed(cbs.at[pl.ds(cur, 16)], ns, mask=ok)
                    plsc.store_compressed(cbd.at[pl.ds(cur, 16)], nd, mask=ok)
                    cur = cur + plsc.all_reduce_population_count(ok)[0]
                return cur

            def row_body(r, cur):
                pltpu.sync_copy(ns_hbm.at[pl.ds(r * 128, 128)],
                                srcb.at[pl.ds(0, 128)])
                pltpu.sync_copy(nd_hbm.at[pl.ds(r * 128, 128)],
                                dstb.at[pl.ds(0, 128)])
                for c in range(8):
                    s16 = srcb[pl.ds(c * 16, 16)]
                    d16 = dstb[pl.ds(c * 16, 16)]
                    ns = plsc.load_gather(mapl, [s16])
                    nd = plsc.load_gather(mapl, [d16])
                    ok = (ns < N2) & (nd < N2)
                    plsc.store_compressed(cbs.at[pl.ds(cur, 16)], ns, mask=ok)
                    plsc.store_compressed(cbd.at[pl.ds(cur, 16)], nd, mask=ok)
                    cur = cur + plsc.all_reduce_population_count(ok)[0]
                return cur

            nfull = (hi - lo) // 8
            cur = lax.fori_loop(0, nfull, batch_body, 0)
            cur = lax.fori_loop(lo + nfull * 8, hi, row_body, cur)
            for j in range(8):
                cbs[pl.ds(cur + j * 16, 16)] = jnp.full((16,), SENT2, I32)
                cbd[pl.ds(cur + j * 16, 16)] = jnp.full((16,), SENT2, I32)
            myrows = (cur + 127) // 128
            ivb[...] = jnp.full((16,), myrows, I32)
            pltpu.sync_copy(ivb.at[pl.ds(0, 8)], cnt_sh.at[pl.ds(sid * 8, 8)])

            def deg_body(kk, _):
                idxb[...] = cbd[pl.ds(kk * 16, 16)]
                pltpu.sync_copy(onesb, deg_sh.at[idxb], add=True)
                return 0
            lax.fori_loop(0, myrows * 8, deg_body, 0)
            plsc.subcore_barrier()

            pltpu.sync_copy(cnt_sh, cntl)
            cnts = plsc.load_gather(cntl, [_iota16() * 8])
            lanes = _iota16()
            rowoff = jnp.sum(jnp.where(lanes < sid, cnts, 0))
            total = jnp.sum(cnts)

            def out_body(r, _):
                pltpu.sync_copy(cbs.at[pl.ds(r * 128, 128)],
                                nsc_hbm.at[pl.ds((rowoff + r) * 128, 128)])
                pltpu.sync_copy(cbd.at[pl.ds(r * 128, 128)],
                                ndc_hbm.at[pl.ds((rowoff + r) * 128, 128)])
                return 0
            lax.fori_loop(0, myrows, out_body, 0)

            @pl.when(sid == 0)
            def _():
                ivb[...] = jnp.full((16,), total, I32)
                pltpu.sync_copy(ivb.at[pl.ds(0, 8)], rc2_hbm)
            plsc.subcore_barrier()

            nbase = sid * 160
            pltpu.sync_copy(deg_sh.at[pl.ds(nbase, 160)], fb1)

            @pl.loop(0, 10)
            def _(i):
                s = pl.ds(i * 16, 16)
                fb1[s] = _bitrsqrt(fb1[s] + 1.0)
            pltpu.sync_copy(fb1, dis3_hbm.at[pl.ds(nbase, 160)])

            pltpu.sync_copy(perm_sh.at[pl.ds(nbase, 160)], pb)
            pltpu.async_copy(sc_hbm.at[pb], fb2, sem).wait()
            pltpu.sync_copy(fb2, sv_hbm.at[pl.ds(nbase, 160)])
            pltpu.async_copy(h2_hbm.at[pb], rowb, sem).wait()
            pltpu.sync_copy(rowb, hsel_hbm.at[pl.ds(nbase, 160)])

    return k(map2, ns1d, nd1d, rcnt, h2, score2)


def _tc1(dis, tacc, xp, W1, b1, Ws1):
    def body(dis_r, tacc_r, x_r, w1_r, b1_r, ws1_r, t_r, ss1_r):
        t = dis_r[...] * (tacc_r[...] + dis_r[...] * x_r[...])
        m = jax.nn.relu(t * w1_r[...] + b1_r[...])
        s1 = jnp.dot(m, ws1_r[...], preferred_element_type=F32,
                     precision=HIGHEST)
        t_r[...] = t
        ss1_r[...] = dis_r[...] * s1

    g = NP // 256
    bs = pl.BlockSpec((256, 1), lambda i: (i, 0))

    def full(a):
        return pl.BlockSpec(a.shape, lambda i: (0,) * a.ndim)

    return pl.pallas_call(
        body, grid=(g,),
        in_specs=[bs, bs, bs, full(W1), full(b1), full(Ws1)],
        out_specs=[bs, bs],
        out_shape=[jax.ShapeDtypeStruct((NP, 1), F32),
                   jax.ShapeDtypeStruct((NP, 1), F32)],
    )(dis, tacc, xp, W1, b1, Ws1)


def _tc_rank(score_bn, npg, kk, dummy):
    def body(sc_r, map_r):
        sc = sc_r[...]
        a = sc[:, :, None]
        bt = sc[:, None, :]
        gt = (bt > a).astype(F32)
        ii = lax.broadcasted_iota(I32, (B, npg, npg), 1)
        jj = lax.broadcasted_iota(I32, (B, npg, npg), 2)
        eq = ((bt == a) & (jj < ii)).astype(F32)
        rank = jnp.sum(gt + eq, axis=2).astype(I32)
        g = lax.broadcasted_iota(I32, (B, npg), 0)
        map_r[...] = jnp.where(rank < kk, g * kk + rank, dummy)

    return pl.pallas_call(
        body,
        in_specs=[pl.BlockSpec((B, npg), lambda: (0, 0))],
        out_specs=pl.BlockSpec((B, npg), lambda: (0, 0)),
        out_shape=jax.ShapeDtypeStruct((B, npg), I32),
    )(score_bn)


def _tc2(tvals, svals, dis2, W1, b1, W2, bs1):
    def body(tv_r, sv_r, d2_r, w1_r, b1_r, w2_r, bs1_r, hh_r):
        sv = sv_r[...] + bs1_r[0, 0]
        xk = jax.nn.relu(tv_r[...] * w1_r[...] + b1_r[...]) * jnp.tanh(sv)
        h2pre = jnp.dot(xk, w2_r[...], preferred_element_type=F32,
                        precision=HIGHEST)
        hh_r[...] = d2_r[...] * h2pre

    g = NP1 // 256
    bs = pl.BlockSpec((256, 1), lambda i: (i, 0))

    def full(a):
        return pl.BlockSpec(a.shape, lambda i: (0,) * a.ndim)

    return pl.pallas_call(
        body, grid=(g,),
        in_specs=[bs, bs, bs, full(W1), full(b1), full(W2), full(bs1)],
        out_specs=pl.BlockSpec((256, H), lambda i: (i, 0)),
        out_shape=jax.ShapeDtypeStruct((NP1, H), F32),
    )(tvals, svals, dis2, W1, b1, W2, bs1)


def _tc3(wacc, hh, dis2, b2, Ws2):
    def body(wa_r, wb_r, hh_r, d2_r, b2_r, ws2_r, h2_r, ss2_r):
        accraw = wa_r[...] + wb_r[...]
        agg2 = d2_r[...] * (accraw + hh_r[...]) + b2_r[...]
        h2 = jax.nn.relu(agg2)
        s2 = jnp.dot(h2, ws2_r[...], preferred_element_type=F32,
                     precision=HIGHEST)
        h2_r[...] = h2
        ss2_r[...] = d2_r[...] * s2

    g = NP1 // 256
    bw = pl.BlockSpec((256, H), lambda i: (i, 0))
    bwb = pl.BlockSpec((256, H), lambda i: (i + NP1 // 256, 0))
    bs = pl.BlockSpec((256, 1), lambda i: (i, 0))

    def full(a):
        return pl.BlockSpec(a.shape, lambda i: (0,) * a.ndim)

    return pl.pallas_call(
        body, grid=(g,),
        in_specs=[bw, bwb, bw, bs, full(b2), full(Ws2)],
        out_specs=[bw, bs],
        out_shape=[jax.ShapeDtypeStruct((NP1, H), F32),
                   jax.ShapeDtypeStruct((NP1, 1), F32)],
    )(wacc, wacc, hh, dis2, b2, Ws2)


def _tc4(h2sel, svals2, dis3, W3, bs2):
    def body(hs_r, sv_r, d3_r, w3_r, bs2_r, hh_r):
        sv = sv_r[...] + bs2_r[0, 0]
        xk2 = hs_r[...] * jnp.tanh(sv)
        h3pre = jnp.dot(xk2, w3_r[...], preferred_element_type=F32,
                        precision=HIGHEST)
        hh_r[...] = d3_r[...] * h3pre

    g = NP2 // 256
    bw = pl.BlockSpec((256, H), lambda i: (i, 0))
    bs = pl.BlockSpec((256, 1), lambda i: (i, 0))

    def full(a):
        return pl.BlockSpec(a.shape, lambda i: (0,) * a.ndim)

    return pl.pallas_call(
        body, grid=(g,),
        in_specs=[bw, bs, bs, full(W3), full(bs2)],
        out_specs=bw,
        out_shape=jax.ShapeDtypeStruct((NP2, H), F32),
    )(h2sel, svals2, dis3, W3, bs2)


def _tc_lstm(x2t, wih_t, whh_t, bias, wl_t, bl):
    T = x2t.shape[0]
    BP = x2t.shape[1]

    def body(x_r, wih_r, whh_r, b_r, wl_r, bl_r, out_r):
        whh = whh_r[...]
        wih = wih_r[...]
        bb = b_r[...]

        def step(t, hc):
            h, c = hc
            xt = x_r[t]
            gates = (jnp.dot(xt, wih, preferred_element_type=F32,
                             precision=HIGHEST)
                     + jnp.dot(h, whh, preferred_element_type=F32,
                               precision=HIGHEST) + bb)
            i = jax.nn.sigmoid(gates[:, 0:H])
            f = jax.nn.sigmoid(gates[:, H:2 * H])
            g = jnp.tanh(gates[:, 2 * H:3 * H])
            o = jax.nn.sigmoid(gates[:, 3 * H:4 * H])
            c = f * c + i * g
            h = o * jnp.tanh(c)
            return (h, c)

        h0 = jnp.zeros((BP, H), F32)
        h, _ = lax.fori_loop(0, T, step, (h0, h0))
        out_r[...] = jnp.dot(h, wl_r[...], preferred_element_type=F32,
                             precision=HIGHEST) + bl_r[...]

    def full(a):
        return pl.BlockSpec(a.shape, lambda: (0,) * a.ndim)

    return pl.pallas_call(
        body,
        in_specs=[full(x2t), full(wih_t), full(whh_t), full(bias),
                  full(wl_t), full(bl)],
        out_specs=pl.BlockSpec((BP, H), lambda: (0, 0)),
        out_shape=jax.ShapeDtypeStruct((BP, H), F32),
    )(x2t, wih_t, whh_t, bias, wl_t, bl)


def _tc5(wacc3, hh3, dis3, b3, S, x2o, wf_t, bf):
    def body(wa_r, hh_r, d3_r, b3_r, s_r, x2o_r, wf_r, bf_r, out_r):
        accraw = wa_r[0] + wa_r[1]
        h3 = d3_r[...] * (accraw + hh_r[...]) + b3_r[...]
        x1 = jnp.dot(s_r[...], h3, preferred_element_type=F32,
                     precision=HIGHEST)
        xc = jnp.concatenate([x1, x2o_r[...]], axis=1)
        out_r[...] = jnp.dot(xc, wf_r[...], preferred_element_type=F32,
                             precision=HIGHEST) + bf_r[...]

    def full(a):
        return pl.BlockSpec(a.shape, lambda: (0,) * a.ndim)

    bw = pl.BlockSpec((256, H), lambda: (0, 0))
    bwb = pl.BlockSpec((256, H), lambda: (NP1 // 256, 0))
    del bw, bwb
    return pl.pallas_call(
        body,
        in_specs=[pl.BlockSpec((2, NP2, H), lambda: (0, 0, 0)),
                  full(hh3),
                  pl.BlockSpec((NP2, 1), lambda: (0, 0)),
                  full(b3), full(S), full(x2o), full(wf_t), full(bf)],
        out_specs=pl.BlockSpec((64, 128), lambda: (0, 0)),
        out_shape=jax.ShapeDtypeStruct((64, 128), F32),
    )(wacc3, hh3, dis3, b3, S, x2o, wf_t, bf)


def kernel(x, edge_index, batch, x2, W1, b1, Ws1, bs1, W2, b2, Ws2, bs2,
           W3, b3, Wih, Whh, bih, bhh, Wl, bl, Wf, bf):
    del batch
    xp = jnp.zeros((NP,), F32).at[:N].set(x[:, 0])
    src = edge_index[0]
    dst = edge_index[1]
    padi = jnp.full((EROWS * 128 - E,), EPAD_IDX, I32)
    src1d = jnp.concatenate([src, padi])
    dst1d = jnp.concatenate([dst, padi])
    zw2 = jnp.zeros((NP1 * 16,), F32)
    zw3 = jnp.zeros((NP2 * 16,), F32)
    b1r = b1.reshape(1, H)
    b2r = b2.reshape(1, H)
    b3r = b3.reshape(1, H)
    bs1r = bs1.reshape(1, 1)
    bs2r = bs2.reshape(1, 1)

    dis, tacc = _sc1(xp, src1d, dst1d)
    t2d, ss12d = _tc1(dis.reshape(NP, 1), tacc.reshape(NP, 1),
                      xp.reshape(NP, 1), W1, b1r, Ws1)
    t = t2d.reshape(NP)
    ss1 = ss12d.reshape(NP)

    score = _scs1(ss1, dis, src1d, dst1d)
    map1 = _tc_rank(score[:N].reshape(B, N1G), N1G, K1, N1)
    map1 = jnp.concatenate([map1.reshape(N),
                            jnp.full((NP - N,), N1, I32)])

    nsc, ndc, rcnt, dis2, tvals, svals = _scp1(
        map1, src1d, dst1d, t, score)

    hh2 = _tc2(tvals.reshape(NP1, 1), svals.reshape(NP1, 1),
               dis2.reshape(NP1, 1), W1, b1r, W2, bs1r)
    wacc2 = _scw(hh2.reshape(NP1 * H), nsc, ndc, rcnt, zw2, NP1)
    wacc2 = (wacc2.reshape(2, 16, NP1, 16).transpose(0, 2, 1, 3)
             .reshape(2 * NP1, H))
    h2, ss2 = _tc3(wacc2, hh2, dis2.reshape(NP1, 1), b2r, Ws2)

    score2 = _scs2(ss2.reshape(NP1), dis2, nsc, ndc, rcnt)
    map2 = _tc_rank(score2[:N1].reshape(B, K1), K1, K2, N2)
    map2 = jnp.concatenate([map2.reshape(N1),
                            jnp.full((NP1 - N1,), N2, I32)])

    nsc2, ndc2, rcnt2, dis3, h2sel, svals2 = _scp2(
        map2, nsc, ndc, rcnt, h2, score2)

    hh3 = _tc4(h2sel, svals2.reshape(NP2, 1), dis3.reshape(NP2, 1),
               W3, bs2r)
    wacc3 = _scw(hh3.reshape(NP2 * H), nsc2, ndc2, rcnt2, zw3, NP2)
    wacc3 = (wacc3.reshape(2, 16, NP2, 16).transpose(0, 2, 1, 3)
             .reshape(2, NP2, H))

    x2t = jnp.zeros((x2.shape[1], 64, 8), F32).at[:, :B, :6].set(
        jnp.swapaxes(x2, 0, 1))
    wih_t = jnp.zeros((8, 4 * H), F32).at[:6, :].set(Wih.T)
    bias = (bih + bhh).reshape(1, 4 * H)
    x2o = _tc_lstm(x2t, wih_t, Whh.T, bias, Wl.T, bl.reshape(1, H))

    S = jnp.zeros((64, NP2), F32).at[
        jnp.repeat(jnp.arange(B), K2), jnp.arange(N2)].set(1.0 / K2)
    wf_t = jnp.zeros((2 * H, 128), F32).at[:, :2].set(Wf.T)
    bfp = jnp.zeros((1, 128), F32).at[0, :2].set(bf)
    out = _tc5(wacc3, hh3, dis3.reshape(NP2, 1), b3r, S, x2o, wf_t, bfp)
    return out[:B, :2]


# speedup vs baseline: 1.2225x; 1.2225x over previous
import dataclasses
import functools

import jax
import jax.numpy as jnp
from jax import lax
from jax.experimental import pallas as pl
from jax.experimental.pallas import tpu as pltpu
from jax.experimental.pallas import tpu_sc as plsc

F32 = jnp.float32
I32 = jnp.int32

N = 10000
E = 160000
B = 50
H = 256
N1G = 200
K1 = 100
N1 = B * K1
K2 = 50
N2 = B * K2
NP = 10240
NP1 = 5120
NP2 = 2560
EROWS = 1280
EPAD_IDX = 10100
CAP1 = 16 * 10368
CAP2 = 16 * 10496
SENT1 = N1
SENT2 = N2
HIGHEST = jax.lax.Precision.HIGHEST


def _sc_params():
    cp = pltpu.CompilerParams()
    if "needs_layout_passes" in pltpu.CompilerParams.__dataclass_fields__:
        cp = dataclasses.replace(cp, needs_layout_passes=False)
    return cp


_MESH = plsc.VectorSubcoreMesh(core_axis_name="c", subcore_axis_name="s")


def _bitrsqrt(x):
    i = plsc.bitcast(x, I32)
    i = I32(0x5F3759DF) - (i >> 1)
    y = plsc.bitcast(i, F32)
    for _ in range(3):
        y = y * (1.5 - 0.5 * x * y * y)
    return y


def _iota16():
    return lax.iota(I32, 16)


def _sc1(xp, src1d, dst1d):
    @functools.partial(
        pl.kernel,
        out_type=[jax.ShapeDtypeStruct((NP,), F32),
                  jax.ShapeDtypeStruct((NP,), F32)],
        mesh=_MESH, compiler_params=_sc_params(),
        scratch_types=[
            pltpu.VMEM_SHARED((NP,), F32),
            pltpu.VMEM((NP,), F32),
            pltpu.VMEM((NP,), F32),
            pltpu.VMEM((1024,), I32),
            pltpu.VMEM((1024,), I32),
            pltpu.VMEM((1024,), F32),
        ] + [pltpu.VMEM((128,), I32) for _ in range(8)] + [
            pltpu.VMEM((128,), F32),
            pltpu.VMEM((640,), F32),
            pltpu.SemaphoreType.DMA,
        ],
    )
    def k(x_hbm, src_hbm, dst_hbm, dis_hbm, tacc_hbm,
          acc_sh, xl, dl, srcb, dstb, valb,
          d0, d1, d2, d3, d4, d5, d6, d7, onesr, zb, sem):
        cid = lax.axis_index("c")
        sid = lax.axis_index("s")
        drows = [d0, d1, d2, d3, d4, d5, d6, d7]

        @pl.when(cid == 0)
        def _():
            @pl.loop(0, 40)
            def _(i):
                zb[pl.ds(i * 16, 16)] = jnp.zeros((16,), F32)
            pltpu.sync_copy(zb, acc_sh.at[pl.ds(sid * 640, 640)])
            pltpu.sync_copy(x_hbm, xl)

            @pl.loop(0, 8)
            def _(i):
                onesr[pl.ds(i * 16, 16)] = jnp.full((16,), 1.0, F32)
            plsc.subcore_barrier()

            @pl.loop(0, 10)
            def _(w):
                base = (sid * 10 + w) * 1024
                pltpu.sync_copy(dst_hbm.at[pl.ds(base, 1024)], dstb)
                for j in range(8):
                    for c in range(8):
                        drows[j][pl.ds(c * 16, 16)] = (
                            dstb[pl.ds(j * 128 + c * 16, 16)])
                cps = [pltpu.async_copy(onesr, acc_sh.at[drows[j]], sem,
                                        add=True) for j in range(8)]
                for cp in cps:
                    cp.wait()
            plsc.subcore_barrier()

            pltpu.sync_copy(acc_sh, dl)

            @pl.loop(0, NP // 16)
            def _(i):
                d16 = dl[pl.ds(i * 16, 16)]
                dl[pl.ds(i * 16, 16)] = _bitrsqrt(d16 + 1.0)
            pltpu.sync_copy(dl.at[pl.ds(sid * 640, 640)],
                            dis_hbm.at[pl.ds(sid * 640, 640)])

            @pl.loop(0, NP // 16)
            def _(i):
                xl[pl.ds(i * 16, 16)] = (xl[pl.ds(i * 16, 16)]
                                         * dl[pl.ds(i * 16, 16)])
            plsc.subcore_barrier()
            pltpu.sync_copy(zb, acc_sh.at[pl.ds(sid * 640, 640)])
            plsc.subcore_barrier()

            @pl.loop(0, 10)
            def _(w):
                base = (sid * 10 + w) * 1024
                pltpu.sync_copy(src_hbm.at[pl.ds(base, 1024)], srcb)
                pltpu.sync_copy(dst_hbm.at[pl.ds(base, 1024)], dstb)
                for j in range(8):
                    for c in range(8):
                        o = j * 128 + c * 16
                        s16 = srcb[pl.ds(o, 16)]
                        valb[pl.ds(o, 16)] = plsc.load_gather(xl, [s16])
                        drows[j][pl.ds(c * 16, 16)] = dstb[pl.ds(o, 16)]
                cps = [pltpu.async_copy(valb.at[pl.ds(j * 128, 128)],
                                        acc_sh.at[drows[j]], sem, add=True)
                       for j in range(8)]
                for cp in cps:
                    cp.wait()
            plsc.subcore_barrier()
            pltpu.sync_copy(acc_sh.at[pl.ds(sid * 640, 640)],
                            tacc_hbm.at[pl.ds(sid * 640, 640)])

    return k(xp, src1d, dst1d)


def _scs1(ss1, dis, src1d, dst1d):
    @functools.partial(
        pl.kernel,
        out_type=jax.ShapeDtypeStruct((NP,), F32),
        mesh=_MESH, compiler_params=_sc_params(),
        scratch_types=[
            pltpu.VMEM_SHARED((NP,), F32),
            pltpu.VMEM((NP,), F32),
            pltpu.VMEM((640,), F32),
            pltpu.VMEM((640,), F32),
            pltpu.VMEM((1024,), I32),
            pltpu.VMEM((1024,), I32),
            pltpu.VMEM((1024,), F32),
        ] + [pltpu.VMEM((128,), I32) for _ in range(8)] + [
            pltpu.VMEM((640,), F32),
            pltpu.SemaphoreType.DMA,
        ],
    )
    def k(ss1_hbm, dis_hbm, src_hbm, dst_hbm, sc_hbm,
          acc_sh, sl, disl, sal, srcb, dstb, valb,
          d0, d1, d2, d3, d4, d5, d6, d7, zb, sem):
        cid = lax.axis_index("c")
        sid = lax.axis_index("s")
        drows = [d0, d1, d2, d3, d4, d5, d6, d7]

        @pl.when(cid == 0)
        def _():
            @pl.loop(0, 40)
            def _(i):
                zb[pl.ds(i * 16, 16)] = jnp.zeros((16,), F32)
            pltpu.sync_copy(zb, acc_sh.at[pl.ds(sid * 640, 640)])
            pltpu.sync_copy(ss1_hbm, sl)
            plsc.subcore_barrier()

            @pl.loop(0, 10)
            def _(w):
                base = (sid * 10 + w) * 1024
                pltpu.sync_copy(src_hbm.at[pl.ds(base, 1024)], srcb)
                pltpu.sync_copy(dst_hbm.at[pl.ds(base, 1024)], dstb)
                for j in range(8):
                    for c in range(8):
                        o = j * 128 + c * 16
                        s16 = srcb[pl.ds(o, 16)]
                        valb[pl.ds(o, 16)] = plsc.load_gather(sl, [s16])
                        drows[j][pl.ds(c * 16, 16)] = dstb[pl.ds(o, 16)]
                cps = [pltpu.async_copy(valb.at[pl.ds(j * 128, 128)],
                                        acc_sh.at[drows[j]], sem, add=True)
                       for j in range(8)]
                for cp in cps:
                    cp.wait()
            plsc.subcore_barrier()

            base = sid * 640
            pltpu.sync_copy(dis_hbm.at[pl.ds(base, 640)], disl)
            pltpu.sync_copy(acc_sh.at[pl.ds(base, 640)], sal)

            @pl.loop(0, 40)
            def _(i):
                s = pl.ds(i * 16, 16)
                sg = pl.ds(base + i * 16, 16)
                sal[s] = disl[s] * (sal[s] + sl[sg])
            pltpu.sync_copy(sal, sc_hbm.at[pl.ds(base, 640)])

    return k(ss1, dis, src1d, dst1d)


def _scp1(map1, src1d, dst1d, t, score):
    @functools.partial(
        pl.kernel,
        out_type=[jax.ShapeDtypeStruct((CAP1,), I32),
                  jax.ShapeDtypeStruct((CAP1,), I32),
                  jax.ShapeDtypeStruct((8,), I32),
                  jax.ShapeDtypeStruct((NP1,), F32),
                  jax.ShapeDtypeStruct((NP1,), F32),
                  jax.ShapeDtypeStruct((NP1,), F32)],
        mesh=_MESH, compiler_params=_sc_params(),
        scratch_types=[
            pltpu.VMEM_SHARED((NP1,), I32),
            pltpu.VMEM_SHARED((NP1,), F32),
            pltpu.VMEM_SHARED((128,), I32),
            pltpu.VMEM((NP,), I32),
            pltpu.VMEM((10368,), I32),
            pltpu.VMEM((10368,), I32),
            pltpu.VMEM((1024,), I32),
            pltpu.VMEM((1024,), I32),
            pltpu.VMEM((16,), I32),
            pltpu.VMEM((16,), I32),
            pltpu.VMEM((16,), F32),
            pltpu.VMEM((128,), I32),
            pltpu.VMEM((320,), F32),
            pltpu.VMEM((320,), F32),
            pltpu.VMEM((320,), I32),
            pltpu.VMEM((320,), F32),
            pltpu.VMEM((320,), I32),
            pltpu.SemaphoreType.DMA,
        ],
    )
    def k(map_hbm, src_hbm, dst_hbm, t_hbm, sc_hbm,
          nsc_hbm, ndc_hbm, rc_hbm, dis2_hbm, tv_hbm, sv_hbm,
          perm_sh, deg_sh, cnt_sh, mapl, cbs, cbd, srcb, dstb,
          idxb, ivb, onesb, cntl, fb1, fb2, pb, zbf, zbi, sem):
        cid = lax.axis_index("c")
        sid = lax.axis_index("s")

        @pl.when(cid == 0)
        def _():
            @pl.loop(0, 20)
            def _(i):
                zbf[pl.ds(i * 16, 16)] = jnp.zeros((16,), F32)
                zbi[pl.ds(i * 16, 16)] = jnp.zeros((16,), I32)
            pltpu.sync_copy(zbi, perm_sh.at[pl.ds(sid * 320, 320)])
            pltpu.sync_copy(zbf, deg_sh.at[pl.ds(sid * 320, 320)])
            pltpu.sync_copy(map_hbm, mapl)
            onesb[...] = jnp.full((16,), 1.0, F32)
            plsc.subcore_barrier()

            @pl.loop(0, 40)
            def _(c):
                base = sid * 640 + c * 16
                m16 = mapl[pl.ds(base, 16)]
                keep = m16 < N1
                idxb[...] = jnp.where(keep, m16, N1)
                ivb[...] = jnp.full((16,), base, I32) + _iota16()
                pltpu.sync_copy(ivb, perm_sh.at[idxb])

            def row_body(w, cur):
                base = (sid * 80 + w * 8) * 128
                pltpu.sync_copy(src_hbm.at[pl.ds(base, 1024)], srcb)
                pltpu.sync_copy(dst_hbm.at[pl.ds(base, 1024)], dstb)
                for c in range(64):
                    s16 = srcb[pl.ds(c * 16, 16)]
                    d16 = dstb[pl.ds(c * 16, 16)]
                    ns = plsc.load_gather(mapl, [s16])
                    nd = plsc.load_gather(mapl, [d16])
                    ok = (ns < N1) & (nd < N1) & (s16 < N)
                    plsc.store_compressed(cbs.at[pl.ds(cur, 16)], ns, mask=ok)
                    plsc.store_compressed(cbd.at[pl.ds(cur, 16)], nd, mask=ok)
                    cur = cur + plsc.all_reduce_population_count(ok)[0]
                return cur

            cur = lax.fori_loop(0, 10, row_body, 0)
            for j in range(8):
                cbs[pl.ds(cur + j * 16, 16)] = jnp.full((16,), SENT1, I32)
                cbd[pl.ds(cur + j * 16, 16)] = jnp.full((16,), SENT1, I32)
            myrows = (cur + 127) // 128
            ivb[...] = jnp.full((16,), myrows, I32)
            pltpu.sync_copy(ivb.at[pl.ds(0, 8)], cnt_sh.at[pl.ds(sid * 8, 8)])

            def deg_body(kk, _):
                idxb[...] = cbd[pl.ds(kk * 16, 16)]
                pltpu.sync_copy(onesb, deg_sh.at[idxb], add=True)
                return 0
            lax.fori_loop(0, myrows * 8, deg_body, 0)
            plsc.subcore_barrier()

            pltpu.sync_copy(cnt_sh, cntl)
            cnts = plsc.load_gather(cntl, [_iota16() * 8])
            lanes = _iota16()
            rowoff = jnp.sum(jnp.where(lanes < sid, cnts, 0))
            total = jnp.sum(cnts)

            def out_body(r, _):
                pltpu.sync_copy(cbs.at[pl.ds(r * 128, 128)],
                                nsc_hbm.at[pl.ds((rowoff + r) * 128, 128)])
                pltpu.sync_copy(cbd.at[pl.ds(r * 128, 128)],
                                ndc_hbm.at[pl.ds((rowoff + r) * 128, 128)])
                return 0
            lax.fori_loop(0, myrows, out_body, 0)

            @pl.when(sid == 0)
            def _():
                ivb[...] = jnp.full((16,), total, I32)
                pltpu.sync_copy(ivb.at[pl.ds(0, 8)], rc_hbm)
            plsc.subcore_barrier()

            nbase = sid * 320
            pltpu.sync_copy(deg_sh.at[pl.ds(nbase, 320)], fb1)

            @pl.loop(0, 20)
            def _(i):
                s = pl.ds(i * 16, 16)
                fb1[s] = _bitrsqrt(fb1[s] + 1.0)
            pltpu.sync_copy(fb1, dis2_hbm.at[pl.ds(nbase, 320)])

            pltpu.sync_copy(perm_sh.at[pl.ds(nbase, 320)], pb)
            pltpu.async_copy(t_hbm.at[pb], fb2, sem).wait()
            pltpu.sync_copy(fb2, tv_hbm.at[pl.ds(nbase, 320)])
            pltpu.async_copy(sc_hbm.at[pb], fb2, sem).wait()
            pltpu.sync_copy(fb2, sv_hbm.at[pl.ds(nbase, 320)])

    return k(map1, src1d, dst1d, t, score)


def _scw(hhflat, ns1d, nd1d, rcnt, zflat, npx):
    nflat = npx * 16

    @functools.partial(
        pl.kernel,
        out_type=jax.ShapeDtypeStruct((32 * nflat,), F32),
        mesh=_MESH, compiler_params=_sc_params(),
        scratch_types=[
            pltpu.VMEM((nflat,), F32),
            pltpu.VMEM((128,), I32),
            pltpu.VMEM((128,), I32),
            pltpu.VMEM((2048,), I32),
            pltpu.VMEM((2048,), F32),
            pltpu.VMEM((128,), I32),
            pltpu.VMEM((128,), I32),
            pltpu.VMEM((2048,), I32),
            pltpu.VMEM((2048,), F32),
            pltpu.VMEM((16,), I32),
            pltpu.SemaphoreType.DMA,
            pltpu.SemaphoreType.DMA,
            pltpu.SemaphoreType.DMA,
        ],
    )
    def k(hh_hbm, ns_hbm, nd_hbm, rc_hbm, z_hbm, out_hbm,
          accl, srowa, drowa, giba, ebufa, srowb, drowb, gibb, ebufb,
          cntb, semi, sema, semb, sem_unused=None):
        cid = lax.axis_index("c")
        sid = lax.axis_index("s")
        wid = cid * 16 + sid

        pltpu.sync_copy(rc_hbm, cntb.at[pl.ds(0, 8)])
        r2 = cntb[pl.ds(0, 16)][0]
        pltpu.sync_copy(z_hbm, accl)

        hr = (r2 + 1) // 2
        lo = cid * hr
        hi = jnp.minimum(lo + hr, r2)
        iot = _iota16()
        colbase = sid * 16

        def build(srow, gib):
            def b_body(e, _):
                ef = jnp.full((16,), e, I32)
                sp = plsc.load_gather(srow, [ef])
                gib[pl.ds(e * 16, 16)] = sp * H + colbase + iot
                return 0
            lax.fori_loop(0, 128, b_body, 0, unroll=4)

        def fire(gib, ebuf, sem):
            return [pltpu.async_copy(
                hh_hbm.at[gib.at[pl.ds(kk * 128, 128)]],
                ebuf.at[pl.ds(kk * 128, 128)], sem) for kk in range(16)]

        def accum(drow, ebuf):
            def e_body(e, _):
                ef = jnp.full((16,), e, I32)
                tb = plsc.load_gather(drow, [ef]) * 16 + iot
                v = ebuf[pl.ds(e * 16, 16)]
                plsc.addupdate_scatter(accl, [tb], v)
                return 0
            lax.fori_loop(0, 128, e_body, 0, unroll=4)

        def pair_body(i, _):
            r0 = lo + 2 * i
            ic = [pltpu.async_copy(ns_hbm.at[pl.ds(r0 * 128, 128)], srowa,
                                   semi),
                  pltpu.async_copy(nd_hbm.at[pl.ds(r0 * 128, 128)], drowa,
                                   semi),
                  pltpu.async_copy(ns_hbm.at[pl.ds(r0 * 128 + 128, 128)],
                                   srowb, semi),
                  pltpu.async_copy(nd_hbm.at[pl.ds(r0 * 128 + 128, 128)],
                                   drowb, semi)]
            for cp in ic:
                cp.wait()
            build(srowa, giba)
            ca = fire(giba, ebufa, sema)
            build(srowb, gibb)
            cb = fire(gibb, ebufb, semb)
            for cp in ca:
                cp.wait()
            accum(drowa, ebufa)
            for cp in cb:
                cp.wait()
            accum(drowb, ebufb)
            return 0

        npairs = (hi - lo) // 2
        lax.fori_loop(0, npairs, pair_body, 0)

        def tail_body(r, _):
            pltpu.sync_copy(ns_hbm.at[pl.ds(r * 128, 128)], srowa)
            pltpu.sync_copy(nd_hbm.at[pl.ds(r * 128, 128)], drowa)
            build(srowa, giba)
            for cp in fire(giba, ebufa, sema):
                cp.wait()
            accum(drowa, ebufa)
            return 0
        lax.fori_loop(lo + npairs * 2, hi, tail_body, 0)

        pltpu.sync_copy(accl, out_hbm.at[pl.ds(wid * nflat, nflat)])

    return k(hhflat, ns1d, nd1d, rcnt, zflat)


def _scs2(ss2, dis2, ns1d, nd1d, rcnt):
    @functools.partial(
        pl.kernel,
        out_type=jax.ShapeDtypeStruct((NP1,), F32),
        mesh=_MESH, compiler_params=_sc_params(),
        scratch_types=[
            pltpu.VMEM_SHARED((NP1,), F32),
            pltpu.VMEM((NP1,), F32),
            pltpu.VMEM((320,), F32),
            pltpu.VMEM((320,), F32),
            pltpu.VMEM((128,), I32),
            pltpu.VMEM((128,), I32),
            pltpu.VMEM((128,), F32),
            pltpu.VMEM((16,), I32),
            pltpu.VMEM((320,), F32),
            pltpu.SemaphoreType.DMA,
        ],
    )
    def k(ss_hbm, dis_hbm, ns_hbm, nd_hbm, rc_hbm, sc_hbm,
          acc_sh, sl, disl, sal, srow, drow, vrow, cntb, zb, sem):
        cid = lax.axis_index("c")
        sid = lax.axis_index("s")

        @pl.when(cid == 0)
        def _():
            @pl.loop(0, 20)
            def _(i):
                zb[pl.ds(i * 16, 16)] = jnp.zeros((16,), F32)
            pltpu.sync_copy(zb, acc_sh.at[pl.ds(sid * 320, 320)])
            pltpu.sync_copy(ss_hbm, sl)
            pltpu.sync_copy(rc_hbm, cntb.at[pl.ds(0, 8)])
            r2 = cntb[pl.ds(0, 16)][0]
            plsc.subcore_barrier()

            tr = (r2 + 15) // 16
            lo = sid * tr
            hi = jnp.minimum(lo + tr, r2)

            def row_body(r, _):
                pltpu.sync_copy(ns_hbm.at[pl.ds(r * 128, 128)], srow)
                pltpu.sync_copy(nd_hbm.at[pl.ds(r * 128, 128)], drow)
                for c in range(8):
                    s = pl.ds(c * 16, 16)
                    vrow[s] = plsc.load_gather(sl, [srow[s]])
                pltpu.sync_copy(vrow, acc_sh.at[drow], add=True)
                return 0
            lax.fori_loop(lo, hi, row_body, 0)
            plsc.subcore_barrier()

            base = sid * 320
            pltpu.sync_copy(dis_hbm.at[pl.ds(base, 320)], disl)
            pltpu.sync_copy(acc_sh.at[pl.ds(base, 320)], sal)

            @pl.loop(0, 20)
            def _(i):
                s = pl.ds(i * 16, 16)
                sg = pl.ds(base + i * 16, 16)
                sal[s] = disl[s] * (sal[s] + sl[sg])
            pltpu.sync_copy(sal, sc_hbm.at[pl.ds(base, 320)])

    return k(ss2, dis2, ns1d, nd1d, rcnt)


def _scp2(map2, ns1d, nd1d, rcnt, h2, score2):
    @functools.partial(
        pl.kernel,
        out_type=[jax.ShapeDtypeStruct((CAP2,), I32),
                  jax.ShapeDtypeStruct((CAP2,), I32),
                  jax.ShapeDtypeStruct((8,), I32),
                  jax.ShapeDtypeStruct((NP2,), F32),
                  jax.ShapeDtypeStruct((NP2, H), F32),
                  jax.ShapeDtypeStruct((NP2,), F32)],
        mesh=_MESH, compiler_params=_sc_params(),
        scratch_types=[
            pltpu.VMEM_SHARED((NP2,), I32),
            pltpu.VMEM_SHARED((NP2,), F32),
            pltpu.VMEM_SHARED((128,), I32),
            pltpu.VMEM((NP1,), I32),
            pltpu.VMEM((10496,), I32),
            pltpu.VMEM((10496,), I32),
            pltpu.VMEM((1024,), I32),
            pltpu.VMEM((1024,), I32),
            pltpu.VMEM((16,), I32),
            pltpu.VMEM((16,), I32),
            pltpu.VMEM((16,), F32),
            pltpu.VMEM((128,), I32),
            pltpu.VMEM((160,), F32),
            pltpu.VMEM((160,), F32),
            pltpu.VMEM((160,), I32),
            pltpu.VMEM((160, H), F32),
            pltpu.VMEM((160,), F32),
            pltpu.VMEM((160,), I32),
            pltpu.SemaphoreType.DMA,
        ],
    )
    def k(map_hbm, ns_hbm, nd_hbm, rc_hbm, h2_hbm, sc_hbm,
          nsc_hbm, ndc_hbm, rc2_hbm, dis3_hbm, hsel_hbm, sv_hbm,
          perm_sh, deg_sh, cnt_sh, mapl, cbs, cbd, srcb, dstb,
          idxb, ivb, onesb, cntl, fb1, fb2, pb, rowb, zbf, zbi, sem):
        cid = lax.axis_index("c")
        sid = lax.axis_index("s")

        @pl.when(cid == 0)
        def _():
            @pl.loop(0, 10)
            def _(i):
                zbf[pl.ds(i * 16, 16)] = jnp.zeros((16,), F32)
                zbi[pl.ds(i * 16, 16)] = jnp.zeros((16,), I32)
            pltpu.sync_copy(zbi, perm_sh.at[pl.ds(sid * 160, 160)])
            pltpu.sync_copy(zbf, deg_sh.at[pl.ds(sid * 160, 160)])
            pltpu.sync_copy(map_hbm, mapl)
            pltpu.sync_copy(rc_hbm, cntl.at[pl.ds(0, 8)])
            r2in = cntl[pl.ds(0, 16)][0]
            onesb[...] = jnp.full((16,), 1.0, F32)
            plsc.subcore_barrier()

            @pl.loop(0, 20)
            def _(c):
                base = sid * 320 + c * 16
                m16 = mapl[pl.ds(base, 16)]
                keep = m16 < N2
                idxb[...] = jnp.where(keep, m16, N2)
                ivb[...] = jnp.full((16,), base, I32) + _iota16()
                pltpu.sync_copy(ivb, perm_sh.at[idxb])

            tr = (r2in + 15) // 16
            lo = sid * tr
            hi = jnp.minimum(lo + tr, r2in)

            def batch_body(w, cur):
                base = (lo + w * 8) * 128
                pltpu.sync_copy(ns_hbm.at[pl.ds(base, 1024)], srcb)
                pltpu.sync_copy(nd_hbm.at[pl.ds(base, 1024)], dstb)
                for c in range(64):
                    s16 = srcb[pl.ds(c * 16, 16)]
                    d16 = dstb[pl.ds(c * 16, 16)]
                    ns = plsc.load_gather(mapl, [s16])
                    nd = plsc.load_gather(mapl, [d16])
                    ok = (ns < N2) & (nd < N2)
                    plsc.store_compressed(cbs.at[pl.ds(cur, 16)], ns, mask=ok)
                    plsc.store_compressed(cbd.at[pl.ds(cur, 16)], nd, mask=ok)
                    cur = cur + plsc.all_reduce_population_count(ok)[0]
                return cur

            def row_body(r, cur):
                pltpu.sync_copy(ns_hbm.at[pl.ds(r * 128, 128)],
                                srcb.at[pl.ds(0, 128)])
                pltpu.sync_copy(nd_hbm.at[pl.ds(r * 128, 128)],
                                dstb.at[pl.ds(0, 128)])
                for c in range(8):
                    s16 = srcb[pl.ds(c * 16, 16)]
                    d16 = dstb[pl.ds(c * 16, 16)]
                    ns = plsc.load_gather(mapl, [s16])
                    nd = plsc.load_gather(mapl, [d16])
                    ok = (ns < N2) & (nd < N2)
                    plsc.store_compressed(cbs.at[pl.ds(cur, 16)], ns, mask=ok)
                    plsc.store_compressed(cbd.at[pl.ds(cur, 16)], nd, mask=ok)
                    cur = cur + plsc.all_reduce_population_count(ok)[0]
                return cur

            nfull = (hi - lo) // 8
            cur = lax.fori_loop(0, nfull, batch_body, 0)
            cur = lax.fori_loop(lo + nfull * 8, hi, row_body, cur)
            for j in range(8):
                cbs[pl.ds(cur + j * 16, 16)] = jnp.full((16,), SENT2, I32)
                cbd[pl.ds(cur + j * 16, 16)] = jnp.full((16,), SENT2, I32)
            myrows = (cur + 127) // 128
            ivb[...] = jnp.full((16,), myrows, I32)
            pltpu.sync_copy(ivb.at[pl.ds(0, 8)], cnt_sh.at[pl.ds(sid * 8, 8)])

            def deg_body(kk, _):
                idxb[...] = cbd[pl.ds(kk * 16, 16)]
                pltpu.sync_copy(onesb, deg_sh.at[idxb], add=True)
                return 0
            lax.fori_loop(0, myrows * 8, deg_body, 0)
            plsc.subcore_barrier()

            pltpu.sync_copy(cnt_sh, cntl)
            cnts = plsc.load_gather(cntl, [_iota16() * 8])
            lanes = _iota16()
            rowoff = jnp.sum(jnp.where(lanes < sid, cnts, 0))
            total = jnp.sum(cnts)

            def out_body(r, _):
                pltpu.sync_copy(cbs.at[pl.ds(r * 128, 128)],
                                nsc_hbm.at[pl.ds((rowoff + r) * 128, 128)])
                pltpu.sync_copy(cbd.at[pl.ds(r * 128, 128)],
                                ndc_hbm.at[pl.ds((rowoff + r) * 128, 128)])
                return 0
            lax.fori_loop(0, myrows, out_body, 0)

            @pl.when(sid == 0)
            def _():
                ivb[...] = jnp.full((16,), total, I32)
                pltpu.sync_copy(ivb.at[pl.ds(0, 8)], rc2_hbm)
            plsc.subcore_barrier()

            nbase = sid * 160
            pltpu.sync_copy(deg_sh.at[pl.ds(nbase, 160)], fb1)

            @pl.loop(0, 10)
            def _(i):
                s = pl.ds(i * 16, 16)
                fb1[s] = _bitrsqrt(fb1[s] + 1.0)
            pltpu.sync_copy(fb1, dis3_hbm.at[pl.ds(nbase, 160)])

            pltpu.sync_copy(perm_sh.at[pl.ds(nbase, 160)], pb)
            pltpu.async_copy(sc_hbm.at[pb], fb2, sem).wait()
            pltpu.sync_copy(fb2, sv_hbm.at[pl.ds(nbase, 160)])
            pltpu.async_copy(h2_hbm.at[pb], rowb, sem).wait()
            pltpu.sync_copy(rowb, hsel_hbm.at[pl.ds(nbase, 160)])

    return k(map2, ns1d, nd1d, rcnt, h2, score2)


def _tc1(dis, tacc, xp, W1, b1, Ws1):
    def body(dis_r, tacc_r, x_r, w1_r, b1_r, ws1_r, t_r, ss1_r):
        t = dis_r[...] * (tacc_r[...] + dis_r[...] * x_r[...])
        m = jax.nn.relu(t * w1_r[...] + b1_r[...])
        s1 = jnp.dot(m, ws1_r[...], preferred_element_type=F32,
                     precision=HIGHEST)
        t_r[...] = t
        ss1_r[...] = dis_r[...] * s1

    g = NP // 256
    bs = pl.BlockSpec((256, 1), lambda i: (i, 0))

    def full(a):
        return pl.BlockSpec(a.shape, lambda i: (0,) * a.ndim)

    return pl.pallas_call(
        body, grid=(g,),
        in_specs=[bs, bs, bs, full(W1), full(b1), full(Ws1)],
        out_specs=[bs, bs],
        out_shape=[jax.ShapeDtypeStruct((NP, 1), F32),
                   jax.ShapeDtypeStruct((NP, 1), F32)],
    )(dis, tacc, xp, W1, b1, Ws1)


def _tc_rank(score_bn, npg, kk, dummy):
    def body(sc_r, map_r):
        sc = sc_r[...]
        a = sc[:, :, None]
        bt = sc[:, None, :]
        gt = (bt > a).astype(F32)
        ii = lax.broadcasted_iota(I32, (B, npg, npg), 1)
        jj = lax.broadcasted_iota(I32, (B, npg, npg), 2)
        eq = ((bt == a) & (jj < ii)).astype(F32)
        rank = jnp.sum(gt + eq, axis=2).astype(I32)
        g = lax.broadcasted_iota(I32, (B, npg), 0)
        map_r[...] = jnp.where(rank < kk, g * kk + rank, dummy)

    return pl.pallas_call(
        body,
        in_specs=[pl.BlockSpec((B, npg), lambda: (0, 0))],
        out_specs=pl.BlockSpec((B, npg), lambda: (0, 0)),
        out_shape=jax.ShapeDtypeStruct((B, npg), I32),
    )(score_bn)


def _tc2(tvals, svals, dis2, W1, b1, W2, bs1):
    def body(tv_r, sv_r, d2_r, w1_r, b1_r, w2_r, bs1_r, hh_r):
        sv = sv_r[...] + bs1_r[0, 0]
        xk = jax.nn.relu(tv_r[...] * w1_r[...] + b1_r[...]) * jnp.tanh(sv)
        h2pre = jnp.dot(xk, w2_r[...], preferred_element_type=F32,
                        precision=HIGHEST)
        hh_r[...] = d2_r[...] * h2pre

    g = NP1 // 256
    bs = pl.BlockSpec((256, 1), lambda i: (i, 0))

    def full(a):
        return pl.BlockSpec(a.shape, lambda i: (0,) * a.ndim)

    return pl.pallas_call(
        body, grid=(g,),
        in_specs=[bs, bs, bs, full(W1), full(b1), full(W2), full(bs1)],
        out_specs=pl.BlockSpec((256, H), lambda i: (i, 0)),
        out_shape=jax.ShapeDtypeStruct((NP1, H), F32),
    )(tvals, svals, dis2, W1, b1, W2, bs1)


def _tc3(wacc, hh, dis2, b2, Ws2):
    def body(wa_r, wb_r, hh_r, d2_r, b2_r, ws2_r, h2_r, ss2_r):
        accraw = wa_r[...] + wb_r[...]
        agg2 = d2_r[...] * (accraw + hh_r[...]) + b2_r[...]
        h2 = jax.nn.relu(agg2)
        s2 = jnp.dot(h2, ws2_r[...], preferred_element_type=F32,
                     precision=HIGHEST)
        h2_r[...] = h2
        ss2_r[...] = d2_r[...] * s2

    g = NP1 // 256
    bw = pl.BlockSpec((256, H), lambda i: (i, 0))
    bwb = pl.BlockSpec((256, H), lambda i: (i + NP1 // 256, 0))
    bs = pl.BlockSpec((256, 1), lambda i: (i, 0))

    def full(a):
        return pl.BlockSpec(a.shape, lambda i: (0,) * a.ndim)

    return pl.pallas_call(
        body, grid=(g,),
        in_specs=[bw, bwb, bw, bs, full(b2), full(Ws2)],
        out_specs=[bw, bs],
        out_shape=[jax.ShapeDtypeStruct((NP1, H), F32),
                   jax.ShapeDtypeStruct((NP1, 1), F32)],
    )(wacc, wacc, hh, dis2, b2, Ws2)


def _tc4(h2sel, svals2, dis3, W3, bs2):
    def body(hs_r, sv_r, d3_r, w3_r, bs2_r, hh_r):
        sv = sv_r[...] + bs2_r[0, 0]
        xk2 = hs_r[...] * jnp.tanh(sv)
        h3pre = jnp.dot(xk2, w3_r[...], preferred_element_type=F32,
                        precision=HIGHEST)
        hh_r[...] = d3_r[...] * h3pre

    g = NP2 // 256
    bw = pl.BlockSpec((256, H), lambda i: (i, 0))
    bs = pl.BlockSpec((256, 1), lambda i: (i, 0))

    def full(a):
        return pl.BlockSpec(a.shape, lambda i: (0,) * a.ndim)

    return pl.pallas_call(
        body, grid=(g,),
        in_specs=[bw, bs, bs, full(W3), full(bs2)],
        out_specs=bw,
        out_shape=jax.ShapeDtypeStruct((NP2, H), F32),
    )(h2sel, svals2, dis3, W3, bs2)


def _tc_lstm(x2t, wih_t, whh_t, bias, wl_t, bl):
    T = x2t.shape[0]
    BP = x2t.shape[1]

    def body(x_r, wih_r, whh_r, b_r, wl_r, bl_r, out_r):
        whh = whh_r[...]
        wih = wih_r[...]
        bb = b_r[...]

        def step(t, hc):
            h, c = hc
            xt = x_r[t]
            gates = (jnp.dot(xt, wih, preferred_element_type=F32,
                             precision=HIGHEST)
                     + jnp.dot(h, whh, preferred_element_type=F32,
                               precision=HIGHEST) + bb)
            i = jax.nn.sigmoid(gates[:, 0:H])
            f = jax.nn.sigmoid(gates[:, H:2 * H])
            g = jnp.tanh(gates[:, 2 * H:3 * H])
            o = jax.nn.sigmoid(gates[:, 3 * H:4 * H])
            c = f * c + i * g
            h = o * jnp.tanh(c)
            return (h, c)

        h0 = jnp.zeros((BP, H), F32)
        h, _ = lax.fori_loop(0, T, step, (h0, h0))
        out_r[...] = jnp.dot(h, wl_r[...], preferred_element_type=F32,
                             precision=HIGHEST) + bl_r[...]

    def full(a):
        return pl.BlockSpec(a.shape, lambda: (0,) * a.ndim)

    return pl.pallas_call(
        body,
        in_specs=[full(x2t), full(wih_t), full(whh_t), full(bias),
                  full(wl_t), full(bl)],
        out_specs=pl.BlockSpec((BP, H), lambda: (0, 0)),
        out_shape=jax.ShapeDtypeStruct((BP, H), F32),
    )(x2t, wih_t, whh_t, bias, wl_t, bl)


def _tc5(wacc3, hh3, dis3, b3, S, x2o, wf_t, bf):
    def body(wa_r, hh_r, d3_r, b3_r, s_r, x2o_r, wf_r, bf_r, out_r):
        accraw = wa_r[0] + wa_r[1]
        h3 = d3_r[...] * (accraw + hh_r[...]) + b3_r[...]
        x1 = jnp.dot(s_r[...], h3, preferred_element_type=F32,
                     precision=HIGHEST)
        xc = jnp.concatenate([x1, x2o_r[...]], axis=1)
        out_r[...] = jnp.dot(xc, wf_r[...], preferred_element_type=F32,
                             precision=HIGHEST) + bf_r[...]

    def full(a):
        return pl.BlockSpec(a.shape, lambda: (0,) * a.ndim)

    bw = pl.BlockSpec((256, H), lambda: (0, 0))
    bwb = pl.BlockSpec((256, H), lambda: (NP1 // 256, 0))
    del bw, bwb
    return pl.pallas_call(
        body,
        in_specs=[pl.BlockSpec((2, NP2, H), lambda: (0, 0, 0)),
                  full(hh3),
                  pl.BlockSpec((NP2, 1), lambda: (0, 0)),
                  full(b3), full(S), full(x2o), full(wf_t), full(bf)],
        out_specs=pl.BlockSpec((64, 128), lambda: (0, 0)),
        out_shape=jax.ShapeDtypeStruct((64, 128), F32),
    )(wacc3, hh3, dis3, b3, S, x2o, wf_t, bf)


def kernel(x, edge_index, batch, x2, W1, b1, Ws1, bs1, W2, b2, Ws2, bs2,
           W3, b3, Wih, Whh, bih, bhh, Wl, bl, Wf, bf):
    del batch
    xp = jnp.zeros((NP,), F32).at[:N].set(x[:, 0])
    src = edge_index[0]
    dst = edge_index[1]
    padi = jnp.full((EROWS * 128 - E,), EPAD_IDX, I32)
    src1d = jnp.concatenate([src, padi])
    dst1d = jnp.concatenate([dst, padi])
    zw2 = jnp.zeros((NP1 * 16,), F32)
    zw3 = jnp.zeros((NP2 * 16,), F32)
    b1r = b1.reshape(1, H)
    b2r = b2.reshape(1, H)
    b3r = b3.reshape(1, H)
    bs1r = bs1.reshape(1, 1)
    bs2r = bs2.reshape(1, 1)

    dis, tacc = _sc1(xp, src1d, dst1d)
    t2d, ss12d = _tc1(dis.reshape(NP, 1), tacc.reshape(NP, 1),
                      xp.reshape(NP, 1), W1, b1r, Ws1)
    t = t2d.reshape(NP)
    ss1 = ss12d.reshape(NP)

    score = _scs1(ss1, dis, src1d, dst1d)
    map1 = _tc_rank(score[:N].reshape(B, N1G), N1G, K1, N1)
    map1 = jnp.concatenate([map1.reshape(N),
                            jnp.full((NP - N,), N1, I32)])

    nsc, ndc, rcnt, dis2, tvals, svals = _scp1(
        map1, src1d, dst1d, t, score)

    hh2 = _tc2(tvals.reshape(NP1, 1), svals.reshape(NP1, 1),
               dis2.reshape(NP1, 1), W1, b1r, W2, bs1r)
    wacc2 = _scw(hh2.reshape(NP1 * H), nsc, ndc, rcnt, zw2, NP1)
    wacc2 = (wacc2.reshape(2, 16, NP1, 16).transpose(0, 2, 1, 3)
             .reshape(2 * NP1, H))
    h2, ss2 = _tc3(wacc2, hh2, dis2.reshape(NP1, 1), b2r, Ws2)

    score2 = _scs2(ss2.reshape(NP1), dis2, nsc, ndc, rcnt)
    map2 = _tc_rank(score2[:N1].reshape(B, K1), K1, K2, N2)
    map2 = jnp.concatenate([map2.reshape(N1),
                            jnp.full((NP1 - N1,), N2, I32)])

    nsc2, ndc2, rcnt2, dis3, h2sel, svals2 = _scp2(
        map2, nsc, ndc, rcnt, h2, score2)

    hh3 = _tc4(h2sel, svals2.reshape(NP2, 1), dis3.reshape(NP2, 1),
               W3, bs2r)
    wacc3 = _scw(hh3.reshape(NP2 * H), nsc2, ndc2, rcnt2, zw3, NP2)
    wacc3 = (wacc3.reshape(2, 16, NP2, 16).transpose(0, 2, 1, 3)
             .reshape(2, NP2, H))

    x2t = jnp.zeros((x2.shape[1], 64, 8), F32).at[:, :B, :6].set(
        jnp.swapaxes(x2, 0, 1))
    wih_t = jnp.zeros((8, 4 * H), F32).at[:6, :].set(Wih.T)
    bias = (bih + bhh).reshape(1, 4 * H)
    x2o = _tc_lstm(x2t, wih_t, Whh.T, bias, Wl.T, bl.reshape(1, H))

    S = jnp.zeros((64, NP2), F32).at[
        jnp.repeat(jnp.arange(B), K2), jnp.arange(N2)].set(1.0 / K2)
    wf_t = jnp.zeros((2 * H, 128), F32).at[:, :2].set(Wf.T)
    bfp = jnp.zeros((1, 128), F32).at[0, :2].set(bf)
    out = _tc5(wacc3, hh3, dis3.reshape(NP2, 1), b3r, S, x2o, wf_t, bfp)
    return out[:B, :2]


# speedup vs baseline: 1.7805x; 1.4564x over previous
import dataclasses
import functools

import jax
import jax.numpy as jnp
from jax import lax
from jax.experimental import pallas as pl
from jax.experimental.pallas import tpu as pltpu
from jax.experimental.pallas import tpu_sc as plsc

F32 = jnp.float32
I32 = jnp.int32

N = 10000
E = 160000
B = 50
H = 256
N1G = 200
K1 = 100
N1 = B * K1
K2 = 50
N2 = B * K2
NP = 10240
NP1 = 5120
NP2 = 2560
EROWS = 1280
EPAD_IDX = 10100
CAP1 = 16 * 10368
CAP2 = 16 * 10496
SENT1 = N1
SENT2 = N2
HIGHEST = jax.lax.Precision.HIGHEST


def _sc_params():
    cp = pltpu.CompilerParams()
    if "needs_layout_passes" in pltpu.CompilerParams.__dataclass_fields__:
        cp = dataclasses.replace(cp, needs_layout_passes=False)
    return cp


_MESH = plsc.VectorSubcoreMesh(core_axis_name="c", subcore_axis_name="s")


def _bitrsqrt(x):
    i = plsc.bitcast(x, I32)
    i = I32(0x5F3759DF) - (i >> 1)
    y = plsc.bitcast(i, F32)
    for _ in range(3):
        y = y * (1.5 - 0.5 * x * y * y)
    return y


def _iota16():
    return lax.iota(I32, 16)


def _sc1(xp, src1d, dst1d):
    @functools.partial(
        pl.kernel,
        out_type=[jax.ShapeDtypeStruct((NP,), F32),
                  jax.ShapeDtypeStruct((NP,), F32)],
        mesh=_MESH, compiler_params=_sc_params(),
        scratch_types=[
            pltpu.VMEM_SHARED((NP,), F32),
            pltpu.VMEM((NP,), F32),
            pltpu.VMEM((NP,), F32),
            pltpu.VMEM((1024,), I32),
            pltpu.VMEM((1024,), I32),
            pltpu.VMEM((1024,), F32),
        ] + [pltpu.VMEM((128,), I32) for _ in range(8)] + [
            pltpu.VMEM((128,), F32),
            pltpu.VMEM((640,), F32),
            pltpu.SemaphoreType.DMA,
        ],
    )
    def k(x_hbm, src_hbm, dst_hbm, dis_hbm, tacc_hbm,
          acc_sh, xl, dl, srcb, dstb, valb,
          d0, d1, d2, d3, d4, d5, d6, d7, onesr, zb, sem):
        cid = lax.axis_index("c")
        sid = lax.axis_index("s")
        drows = [d0, d1, d2, d3, d4, d5, d6, d7]

        @pl.when(cid == 0)
        def _():
            @pl.loop(0, 40)
            def _(i):
                zb[pl.ds(i * 16, 16)] = jnp.zeros((16,), F32)
            pltpu.sync_copy(zb, acc_sh.at[pl.ds(sid * 640, 640)])
            pltpu.sync_copy(x_hbm, xl)

            @pl.loop(0, 8)
            def _(i):
                onesr[pl.ds(i * 16, 16)] = jnp.full((16,), 1.0, F32)
            plsc.subcore_barrier()

            @pl.loop(0, 10)
            def _(w):
                base = (sid * 10 + w) * 1024
                pltpu.sync_copy(dst_hbm.at[pl.ds(base, 1024)], dstb)
                for j in range(8):
                    for c in range(8):
                        drows[j][pl.ds(c * 16, 16)] = (
                            dstb[pl.ds(j * 128 + c * 16, 16)])
                cps = [pltpu.async_copy(onesr, acc_sh.at[drows[j]], sem,
                                        add=True) for j in range(8)]
                for cp in cps:
                    cp.wait()
            plsc.subcore_barrier()

            pltpu.sync_copy(acc_sh, dl)

            @pl.loop(0, NP // 16)
            def _(i):
                d16 = dl[pl.ds(i * 16, 16)]
                dl[pl.ds(i * 16, 16)] = _bitrsqrt(d16 + 1.0)
            pltpu.sync_copy(dl.at[pl.ds(sid * 640, 640)],
                            dis_hbm.at[pl.ds(sid * 640, 640)])

            @pl.loop(0, NP // 16)
            def _(i):
                xl[pl.ds(i * 16, 16)] = (xl[pl.ds(i * 16, 16)]
                                         * dl[pl.ds(i * 16, 16)])
            plsc.subcore_barrier()
            pltpu.sync_copy(zb, acc_sh.at[pl.ds(sid * 640, 640)])
            plsc.subcore_barrier()

            @pl.loop(0, 10)
            def _(w):
                base = (sid * 10 + w) * 1024
                pltpu.sync_copy(src_hbm.at[pl.ds(base, 1024)], srcb)
                pltpu.sync_copy(dst_hbm.at[pl.ds(base, 1024)], dstb)
                for j in range(8):
                    for c in range(8):
                        o = j * 128 + c * 16
                        s16 = srcb[pl.ds(o, 16)]
                        valb[pl.ds(o, 16)] = plsc.load_gather(xl, [s16])
                        drows[j][pl.ds(c * 16, 16)] = dstb[pl.ds(o, 16)]
                cps = [pltpu.async_copy(valb.at[pl.ds(j * 128, 128)],
                                        acc_sh.at[drows[j]], sem, add=True)
                       for j in range(8)]
                for cp in cps:
                    cp.wait()
            plsc.subcore_barrier()
            pltpu.sync_copy(acc_sh.at[pl.ds(sid * 640, 640)],
                            tacc_hbm.at[pl.ds(sid * 640, 640)])

    return k(xp, src1d, dst1d)


def _scs1(ss1, dis, src1d, dst1d):
    @functools.partial(
        pl.kernel,
        out_type=jax.ShapeDtypeStruct((NP,), F32),
        mesh=_MESH, compiler_params=_sc_params(),
        scratch_types=[
            pltpu.VMEM_SHARED((NP,), F32),
            pltpu.VMEM((NP,), F32),
            pltpu.VMEM((640,), F32),
            pltpu.VMEM((640,), F32),
            pltpu.VMEM((1024,), I32),
            pltpu.VMEM((1024,), I32),
            pltpu.VMEM((1024,), F32),
        ] + [pltpu.VMEM((128,), I32) for _ in range(8)] + [
            pltpu.VMEM((640,), F32),
            pltpu.SemaphoreType.DMA,
        ],
    )
    def k(ss1_hbm, dis_hbm, src_hbm, dst_hbm, sc_hbm,
          acc_sh, sl, disl, sal, srcb, dstb, valb,
          d0, d1, d2, d3, d4, d5, d6, d7, zb, sem):
        cid = lax.axis_index("c")
        sid = lax.axis_index("s")
        drows = [d0, d1, d2, d3, d4, d5, d6, d7]

        @pl.when(cid == 0)
        def _():
            @pl.loop(0, 40)
            def _(i):
                zb[pl.ds(i * 16, 16)] = jnp.zeros((16,), F32)
            pltpu.sync_copy(zb, acc_sh.at[pl.ds(sid * 640, 640)])
            pltpu.sync_copy(ss1_hbm, sl)
            plsc.subcore_barrier()

            @pl.loop(0, 10)
            def _(w):
                base = (sid * 10 + w) * 1024
                pltpu.sync_copy(src_hbm.at[pl.ds(base, 1024)], srcb)
                pltpu.sync_copy(dst_hbm.at[pl.ds(base, 1024)], dstb)
                for j in range(8):
                    for c in range(8):
                        o = j * 128 + c * 16
                        s16 = srcb[pl.ds(o, 16)]
                        valb[pl.ds(o, 16)] = plsc.load_gather(sl, [s16])
                        drows[j][pl.ds(c * 16, 16)] = dstb[pl.ds(o, 16)]
                cps = [pltpu.async_copy(valb.at[pl.ds(j * 128, 128)],
                                        acc_sh.at[drows[j]], sem, add=True)
                       for j in range(8)]
                for cp in cps:
                    cp.wait()
            plsc.subcore_barrier()

            base = sid * 640
            pltpu.sync_copy(dis_hbm.at[pl.ds(base, 640)], disl)
            pltpu.sync_copy(acc_sh.at[pl.ds(base, 640)], sal)

            @pl.loop(0, 40)
            def _(i):
                s = pl.ds(i * 16, 16)
                sg = pl.ds(base + i * 16, 16)
                sal[s] = disl[s] * (sal[s] + sl[sg])
            pltpu.sync_copy(sal, sc_hbm.at[pl.ds(base, 640)])

    return k(ss1, dis, src1d, dst1d)


def _scp1(map1, src1d, dst1d, t, score):
    @functools.partial(
        pl.kernel,
        out_type=[jax.ShapeDtypeStruct((CAP1,), I32),
                  jax.ShapeDtypeStruct((CAP1,), I32),
                  jax.ShapeDtypeStruct((8,), I32),
                  jax.ShapeDtypeStruct((NP1,), F32),
                  jax.ShapeDtypeStruct((NP1,), F32),
                  jax.ShapeDtypeStruct((NP1,), F32)],
        mesh=_MESH, compiler_params=_sc_params(),
        scratch_types=[
            pltpu.VMEM_SHARED((NP1,), I32),
            pltpu.VMEM_SHARED((NP1,), F32),
            pltpu.VMEM_SHARED((128,), I32),
            pltpu.VMEM((NP,), I32),
            pltpu.VMEM((10368,), I32),
            pltpu.VMEM((10368,), I32),
            pltpu.VMEM((1024,), I32),
            pltpu.VMEM((1024,), I32),
            pltpu.VMEM((16,), I32),
            pltpu.VMEM((16,), I32),
            pltpu.VMEM((16,), F32),
            pltpu.VMEM((128,), I32),
            pltpu.VMEM((320,), F32),
            pltpu.VMEM((320,), F32),
            pltpu.VMEM((320,), I32),
            pltpu.VMEM((320,), F32),
            pltpu.VMEM((320,), I32),
            pltpu.SemaphoreType.DMA,
        ],
    )
    def k(map_hbm, src_hbm, dst_hbm, t_hbm, sc_hbm,
          nsc_hbm, ndc_hbm, rc_hbm, dis2_hbm, tv_hbm, sv_hbm,
          perm_sh, deg_sh, cnt_sh, mapl, cbs, cbd, srcb, dstb,
          idxb, ivb, onesb, cntl, fb1, fb2, pb, zbf, zbi, sem):
        cid = lax.axis_index("c")
        sid = lax.axis_index("s")

        @pl.when(cid == 0)
        def _():
            @pl.loop(0, 20)
            def _(i):
                zbf[pl.ds(i * 16, 16)] = jnp.zeros((16,), F32)
                zbi[pl.ds(i * 16, 16)] = jnp.zeros((16,), I32)
            pltpu.sync_copy(zbi, perm_sh.at[pl.ds(sid * 320, 320)])
            pltpu.sync_copy(zbf, deg_sh.at[pl.ds(sid * 320, 320)])
            pltpu.sync_copy(map_hbm, mapl)
            onesb[...] = jnp.full((16,), 1.0, F32)
            plsc.subcore_barrier()

            @pl.loop(0, 40)
            def _(c):
                base = sid * 640 + c * 16
                m16 = mapl[pl.ds(base, 16)]
                keep = m16 < N1
                idxb[...] = jnp.where(keep, m16, N1)
                ivb[...] = jnp.full((16,), base, I32) + _iota16()
                pltpu.sync_copy(ivb, perm_sh.at[idxb])

            def row_body(w, cur):
                base = (sid * 80 + w * 8) * 128
                pltpu.sync_copy(src_hbm.at[pl.ds(base, 1024)], srcb)
                pltpu.sync_copy(dst_hbm.at[pl.ds(base, 1024)], dstb)
                for c in range(64):
                    s16 = srcb[pl.ds(c * 16, 16)]
                    d16 = dstb[pl.ds(c * 16, 16)]
                    ns = plsc.load_gather(mapl, [s16])
                    nd = plsc.load_gather(mapl, [d16])
                    ok = (ns < N1) & (nd < N1) & (s16 < N)
                    plsc.store_compressed(cbs.at[pl.ds(cur, 16)], ns, mask=ok)
                    plsc.store_compressed(cbd.at[pl.ds(cur, 16)], nd, mask=ok)
                    cur = cur + plsc.all_reduce_population_count(ok)[0]
                return cur

            cur = lax.fori_loop(0, 10, row_body, 0)
            for j in range(8):
                cbs[pl.ds(cur + j * 16, 16)] = jnp.full((16,), SENT1, I32)
                cbd[pl.ds(cur + j * 16, 16)] = jnp.full((16,), SENT1, I32)
            myrows = (cur + 127) // 128
            ivb[...] = jnp.full((16,), myrows, I32)
            pltpu.sync_copy(ivb.at[pl.ds(0, 8)], cnt_sh.at[pl.ds(sid * 8, 8)])

            def deg_body(kk, _):
                idxb[...] = cbd[pl.ds(kk * 16, 16)]
                pltpu.sync_copy(onesb, deg_sh.at[idxb], add=True)
                return 0
            lax.fori_loop(0, myrows * 8, deg_body, 0)
            plsc.subcore_barrier()

            pltpu.sync_copy(cnt_sh, cntl)
            cnts = plsc.load_gather(cntl, [_iota16() * 8])
            lanes = _iota16()
            rowoff = jnp.sum(jnp.where(lanes < sid, cnts, 0))
            total = jnp.sum(cnts)

            def out_body(r, _):
                pltpu.sync_copy(cbs.at[pl.ds(r * 128, 128)],
                                nsc_hbm.at[pl.ds((rowoff + r) * 128, 128)])
                pltpu.sync_copy(cbd.at[pl.ds(r * 128, 128)],
                                ndc_hbm.at[pl.ds((rowoff + r) * 128, 128)])
                return 0
            lax.fori_loop(0, myrows, out_body, 0)

            @pl.when(sid == 0)
            def _():
                ivb[...] = jnp.full((16,), total, I32)
                pltpu.sync_copy(ivb.at[pl.ds(0, 8)], rc_hbm)
            plsc.subcore_barrier()

            nbase = sid * 320
            pltpu.sync_copy(deg_sh.at[pl.ds(nbase, 320)], fb1)

            @pl.loop(0, 20)
            def _(i):
                s = pl.ds(i * 16, 16)
                fb1[s] = _bitrsqrt(fb1[s] + 1.0)
            pltpu.sync_copy(fb1, dis2_hbm.at[pl.ds(nbase, 320)])

            pltpu.sync_copy(perm_sh.at[pl.ds(nbase, 320)], pb)
            pltpu.async_copy(t_hbm.at[pb], fb2, sem).wait()
            pltpu.sync_copy(fb2, tv_hbm.at[pl.ds(nbase, 320)])
            pltpu.async_copy(sc_hbm.at[pb], fb2, sem).wait()
            pltpu.sync_copy(fb2, sv_hbm.at[pl.ds(nbase, 320)])

    return k(map1, src1d, dst1d, t, score)


def _scw(hhhalf, ns1d, nd1d, rcnt, zflat, npx):
    nflat = npx * 16

    @functools.partial(
        pl.kernel,
        out_type=jax.ShapeDtypeStruct((32 * nflat,), F32),
        mesh=_MESH, compiler_params=_sc_params(),
        scratch_types=[
            pltpu.VMEM((nflat,), F32),
            pltpu.VMEM((128,), I32),
            pltpu.VMEM((128,), I32),
            pltpu.VMEM((128,), I32),
            pltpu.VMEM((128, 128), F32),
            pltpu.VMEM((128,), I32),
            pltpu.VMEM((128,), I32),
            pltpu.VMEM((128,), I32),
            pltpu.VMEM((128, 128), F32),
            pltpu.VMEM((16,), I32),
            pltpu.SemaphoreType.DMA,
            pltpu.SemaphoreType.DMA,
            pltpu.SemaphoreType.DMA,
        ],
    )
    def k(hh_hbm, ns_hbm, nd_hbm, rc_hbm, z_hbm, out_hbm,
          accl, srowa, drowa, giba, rbufa, srowb, drowb, gibb, rbufb,
          cntb, semi, sema, semb):
        cid = lax.axis_index("c")
        sid = lax.axis_index("s")
        wid = cid * 16 + sid

        pltpu.sync_copy(rc_hbm, cntb.at[pl.ds(0, 8)])
        r2 = cntb[pl.ds(0, 16)][0]
        pltpu.sync_copy(z_hbm, accl)

        hr = (r2 + 1) // 2
        lo = cid * hr
        hi = jnp.minimum(lo + hr, r2)
        iot = _iota16()
        half = sid // 8
        csub = (sid % 8) * 16

        def build(srow, gib):
            for c in range(8):
                s = pl.ds(c * 16, 16)
                gib[s] = srow[s] * 2 + half

        def accum(drow, rbuf):
            def e_body(e, _):
                ef = jnp.full((16,), e, I32)
                tb = plsc.load_gather(drow, [ef]) * 16 + iot
                v = plsc.load_gather(rbuf, [ef, csub + iot])
                plsc.addupdate_scatter(accl, [tb], v)
                return 0
            lax.fori_loop(0, 128, e_body, 0, unroll=4)

        def pair_body(i, _):
            r0 = lo + 2 * i
            ic = [pltpu.async_copy(ns_hbm.at[pl.ds(r0 * 128, 128)], srowa,
                                   semi),
                  pltpu.async_copy(nd_hbm.at[pl.ds(r0 * 128, 128)], drowa,
                                   semi),
                  pltpu.async_copy(ns_hbm.at[pl.ds(r0 * 128 + 128, 128)],
                                   srowb, semi),
                  pltpu.async_copy(nd_hbm.at[pl.ds(r0 * 128 + 128, 128)],
                                   drowb, semi)]
            for cp in ic:
                cp.wait()
            build(srowa, giba)
            ca = pltpu.async_copy(hh_hbm.at[giba], rbufa, sema)
            build(srowb, gibb)
            cb = pltpu.async_copy(hh_hbm.at[gibb], rbufb, semb)
            ca.wait()
            accum(drowa, rbufa)
            cb.wait()
            accum(drowb, rbufb)
            return 0

        npairs = (hi - lo) // 2
        lax.fori_loop(0, npairs, pair_body, 0)

        def tail_body(r, _):
            pltpu.sync_copy(ns_hbm.at[pl.ds(r * 128, 128)], srowa)
            pltpu.sync_copy(nd_hbm.at[pl.ds(r * 128, 128)], drowa)
            build(srowa, giba)
            pltpu.async_copy(hh_hbm.at[giba], rbufa, sema).wait()
            accum(drowa, rbufa)
            return 0
        lax.fori_loop(lo + npairs * 2, hi, tail_body, 0)

        pltpu.sync_copy(accl, out_hbm.at[pl.ds(wid * nflat, nflat)])

    return k(hhhalf, ns1d, nd1d, rcnt, zflat)


def _scs2(ss2, dis2, ns1d, nd1d, rcnt):
    @functools.partial(
        pl.kernel,
        out_type=jax.ShapeDtypeStruct((NP1,), F32),
        mesh=_MESH, compiler_params=_sc_params(),
        scratch_types=[
            pltpu.VMEM_SHARED((NP1,), F32),
            pltpu.VMEM((NP1,), F32),
            pltpu.VMEM((320,), F32),
            pltpu.VMEM((320,), F32),
            pltpu.VMEM((128,), I32),
            pltpu.VMEM((128,), I32),
            pltpu.VMEM((128,), F32),
            pltpu.VMEM((16,), I32),
            pltpu.VMEM((320,), F32),
            pltpu.SemaphoreType.DMA,
        ],
    )
    def k(ss_hbm, dis_hbm, ns_hbm, nd_hbm, rc_hbm, sc_hbm,
          acc_sh, sl, disl, sal, srow, drow, vrow, cntb, zb, sem):
        cid = lax.axis_index("c")
        sid = lax.axis_index("s")

        @pl.when(cid == 0)
        def _():
            @pl.loop(0, 20)
            def _(i):
                zb[pl.ds(i * 16, 16)] = jnp.zeros((16,), F32)
            pltpu.sync_copy(zb, acc_sh.at[pl.ds(sid * 320, 320)])
            pltpu.sync_copy(ss_hbm, sl)
            pltpu.sync_copy(rc_hbm, cntb.at[pl.ds(0, 8)])
            r2 = cntb[pl.ds(0, 16)][0]
            plsc.subcore_barrier()

            tr = (r2 + 15) // 16
            lo = sid * tr
            hi = jnp.minimum(lo + tr, r2)

            def row_body(r, _):
                pltpu.sync_copy(ns_hbm.at[pl.ds(r * 128, 128)], srow)
                pltpu.sync_copy(nd_hbm.at[pl.ds(r * 128, 128)], drow)
                for c in range(8):
                    s = pl.ds(c * 16, 16)
                    vrow[s] = plsc.load_gather(sl, [srow[s]])
                pltpu.sync_copy(vrow, acc_sh.at[drow], add=True)
                return 0
            lax.fori_loop(lo, hi, row_body, 0)
            plsc.subcore_barrier()

            base = sid * 320
            pltpu.sync_copy(dis_hbm.at[pl.ds(base, 320)], disl)
            pltpu.sync_copy(acc_sh.at[pl.ds(base, 320)], sal)

            @pl.loop(0, 20)
            def _(i):
                s = pl.ds(i * 16, 16)
                sg = pl.ds(base + i * 16, 16)
                sal[s] = disl[s] * (sal[s] + sl[sg])
            pltpu.sync_copy(sal, sc_hbm.at[pl.ds(base, 320)])

    return k(ss2, dis2, ns1d, nd1d, rcnt)


def _scp2(map2, ns1d, nd1d, rcnt, h2, score2):
    @functools.partial(
        pl.kernel,
        out_type=[jax.ShapeDtypeStruct((CAP2,), I32),
                  jax.ShapeDtypeStruct((CAP2,), I32),
                  jax.ShapeDtypeStruct((8,), I32),
                  jax.ShapeDtypeStruct((NP2,), F32),
                  jax.ShapeDtypeStruct((NP2, H), F32),
                  jax.ShapeDtypeStruct((NP2,), F32)],
        mesh=_MESH, compiler_params=_sc_params(),
        scratch_types=[
            pltpu.VMEM_SHARED((NP2,), I32),
            pltpu.VMEM_SHARED((NP2,), F32),
            pltpu.VMEM_SHARED((128,), I32),
            pltpu.VMEM((NP1,), I32),
            pltpu.VMEM((10496,), I32),
            pltpu.VMEM((10496,), I32),
            pltpu.VMEM((1024,), I32),
            pltpu.VMEM((1024,), I32),
            pltpu.VMEM((16,), I32),
            pltpu.VMEM((16,), I32),
            pltpu.VMEM((16,), F32),
            pltpu.VMEM((128,), I32),
            pltpu.VMEM((160,), F32),
            pltpu.VMEM((160,), F32),
            pltpu.VMEM((160,), I32),
            pltpu.VMEM((160, H), F32),
            pltpu.VMEM((160,), F32),
            pltpu.VMEM((160,), I32),
            pltpu.SemaphoreType.DMA,
        ],
    )
    def k(map_hbm, ns_hbm, nd_hbm, rc_hbm, h2_hbm, sc_hbm,
          nsc_hbm, ndc_hbm, rc2_hbm, dis3_hbm, hsel_hbm, sv_hbm,
          perm_sh, deg_sh, cnt_sh, mapl, cbs, cbd, srcb, dstb,
          idxb, ivb, onesb, cntl, fb1, fb2, pb, rowb, zbf, zbi, sem):
        cid = lax.axis_index("c")
        sid = lax.axis_index("s")

        @pl.when(cid == 0)
        def _():
            @pl.loop(0, 10)
            def _(i):
                zbf[pl.ds(i * 16, 16)] = jnp.zeros((16,), F32)
                zbi[pl.ds(i * 16, 16)] = jnp.zeros((16,), I32)
            pltpu.sync_copy(zbi, perm_sh.at[pl.ds(sid * 160, 160)])
            pltpu.sync_copy(zbf, deg_sh.at[pl.ds(sid * 160, 160)])
            pltpu.sync_copy(map_hbm, mapl)
            pltpu.sync_copy(rc_hbm, cntl.at[pl.ds(0, 8)])
            r2in = cntl[pl.ds(0, 16)][0]
            onesb[...] = jnp.full((16,), 1.0, F32)
            plsc.subcore_barrier()

            @pl.loop(0, 20)
            def _(c):
                base = sid * 320 + c * 16
                m16 = mapl[pl.ds(base, 16)]
                keep = m16 < N2
                idxb[...] = jnp.where(keep, m16, N2)
                ivb[...] = jnp.full((16,), base, I32) + _iota16()
                pltpu.sync_copy(ivb, perm_sh.at[idxb])

            tr = (r2in + 15) // 16
            lo = sid * tr
            hi = jnp.minimum(lo + tr, r2in)

            def batch_body(w, cur):
                base = (lo + w * 8) * 128
                pltpu.sync_copy(ns_hbm.at[pl.ds(base, 1024)], srcb)
                pltpu.sync_copy(nd_hbm.at[pl.ds(base, 1024)], dstb)
                for c in range(64):
                    s16 = srcb[pl.ds(c * 16, 16)]
                    d16 = dstb[pl.ds(c * 16, 16)]
                    ns = plsc.load_gather(mapl, [s16])
                    nd = plsc.load_gather(mapl, [d16])
                    ok = (ns < N2) & (nd < N2)
                    plsc.store_compressed(cbs.at[pl.ds(cur, 16)], ns, mask=ok)
                    plsc.store_compressed(cbd.at[pl.ds(cur, 16)], nd, mask=ok)
                    cur = cur + plsc.all_reduce_population_count(ok)[0]
                return cur

            def row_body(r, cur):
                pltpu.sync_copy(ns_hbm.at[pl.ds(r * 128, 128)],
                                srcb.at[pl.ds(0, 128)])
                pltpu.sync_copy(nd_hbm.at[pl.ds(r * 128, 128)],
                                dstb.at[pl.ds(0, 128)])
                for c in range(8):
                    s16 = srcb[pl.ds(c * 16, 16)]
                    d16 = dstb[pl.ds(c * 16, 16)]
                    ns = plsc.load_gather(mapl, [s16])
                    nd = plsc.load_gather(mapl, [d16])
                    ok = (ns < N2) & (nd < N2)
                    plsc.store_compressed(cbs.at[pl.ds(cur, 16)], ns, mask=ok)
                    plsc.store_compressed(cbd.at[pl.ds(cur, 16)], nd, mask=ok)
                    cur = cur + plsc.all_reduce_population_count(ok)[0]
                return cur

            nfull = (hi - lo) // 8
            cur = lax.fori_loop(0, nfull, batch_body, 0)
            cur = lax.fori_loop(lo + nfull * 8, hi, row_body, cur)
            for j in range(8):
                cbs[pl.ds(cur + j * 16, 16)] = jnp.full((16,), SENT2, I32)
                cbd[pl.ds(cur + j * 16, 16)] = jnp.full((16,), SENT2, I32)
            myrows = (cur + 127) // 128
            ivb[...] = jnp.full((16,), myrows, I32)
            pltpu.sync_copy(ivb.at[pl.ds(0, 8)], cnt_sh.at[pl.ds(sid * 8, 8)])

            def deg_body(kk, _):
                idxb[...] = cbd[pl.ds(kk * 16, 16)]
                pltpu.sync_copy(onesb, deg_sh.at[idxb], add=True)
                return 0
            lax.fori_loop(0, myrows * 8, deg_body, 0)
            plsc.subcore_barrier()

            pltpu.sync_copy(cnt_sh, cntl)
            cnts = plsc.load_gather(cntl, [_iota16() * 8])
            lanes = _iota16()
            rowoff = jnp.sum(jnp.where(lanes < sid, cnts, 0))
            total = jnp.sum(cnts)

            def out_body(r, _):
                pltpu.sync_copy(cbs.at[pl.ds(r * 128, 128)],
                                nsc_hbm.at[pl.ds((rowoff + r) * 128, 128)])
                pltpu.sync_copy(cbd.at[pl.ds(r * 128, 128)],
                                ndc_hbm.at[pl.ds((rowoff + r) * 128, 128)])
                return 0
            lax.fori_loop(0, myrows, out_body, 0)

            @pl.when(sid == 0)
            def _():
                ivb[...] = jnp.full((16,), total, I32)
                pltpu.sync_copy(ivb.at[pl.ds(0, 8)], rc2_hbm)
            plsc.subcore_barrier()

            nbase = sid * 160
            pltpu.sync_copy(deg_sh.at[pl.ds(nbase, 160)], fb1)

            @pl.loop(0, 10)
            def _(i):
                s = pl.ds(i * 16, 16)
                fb1[s] = _bitrsqrt(fb1[s] + 1.0)
            pltpu.sync_copy(fb1, dis3_hbm.at[pl.ds(nbase, 160)])

            pltpu.sync_copy(perm_sh.at[pl.ds(nbase, 160)], pb)
            pltpu.async_copy(sc_hbm.at[pb], fb2, sem).wait()
            pltpu.sync_copy(fb2, sv_hbm.at[pl.ds(nbase, 160)])
            pltpu.async_copy(h2_hbm.at[pb], rowb, sem).wait()
            pltpu.sync_copy(rowb, hsel_hbm.at[pl.ds(nbase, 160)])

    return k(map2, ns1d, nd1d, rcnt, h2, score2)


def _tc1(dis, tacc, xp, W1, b1, Ws1):
    def body(dis_r, tacc_r, x_r, w1_r, b1_r, ws1_r, t_r, ss1_r):
        t = dis_r[...] * (tacc_r[...] + dis_r[...] * x_r[...])
        m = jax.nn.relu(t * w1_r[...] + b1_r[...])
        s1 = jnp.dot(m, ws1_r[...], preferred_element_type=F32,
                     precision=HIGHEST)
        t_r[...] = t
        ss1_r[...] = dis_r[...] * s1

    g = NP // 256
    bs = pl.BlockSpec((256, 1), lambda i: (i, 0))

    def full(a):
        return pl.BlockSpec(a.shape, lambda i: (0,) * a.ndim)

    return pl.pallas_call(
        body, grid=(g,),
        in_specs=[bs, bs, bs, full(W1), full(b1), full(Ws1)],
        out_specs=[bs, bs],
        out_shape=[jax.ShapeDtypeStruct((NP, 1), F32),
                   jax.ShapeDtypeStruct((NP, 1), F32)],
    )(dis, tacc, xp, W1, b1, Ws1)


def _tc_rank(score_bn, npg, kk, dummy):
    def body(sc_r, map_r):
        sc = sc_r[...]
        a = sc[:, :, None]
        bt = sc[:, None, :]
        gt = (bt > a).astype(F32)
        ii = lax.broadcasted_iota(I32, (B, npg, npg), 1)
        jj = lax.broadcasted_iota(I32, (B, npg, npg), 2)
        eq = ((bt == a) & (jj < ii)).astype(F32)
        rank = jnp.sum(gt + eq, axis=2).astype(I32)
        g = lax.broadcasted_iota(I32, (B, npg), 0)
        map_r[...] = jnp.where(rank < kk, g * kk + rank, dummy)

    return pl.pallas_call(
        body,
        in_specs=[pl.BlockSpec((B, npg), lambda: (0, 0))],
        out_specs=pl.BlockSpec((B, npg), lambda: (0, 0)),
        out_shape=jax.ShapeDtypeStruct((B, npg), I32),
    )(score_bn)


def _tc2(tvals, svals, dis2, W1, b1, W2, bs1):
    def body(tv_r, sv_r, d2_r, w1_r, b1_r, w2_r, bs1_r, hh_r):
        sv = sv_r[...] + bs1_r[0, 0]
        xk = jax.nn.relu(tv_r[...] * w1_r[...] + b1_r[...]) * jnp.tanh(sv)
        h2pre = jnp.dot(xk, w2_r[...], preferred_element_type=F32,
                        precision=HIGHEST)
        hh_r[...] = d2_r[...] * h2pre

    g = NP1 // 256
    bs = pl.BlockSpec((256, 1), lambda i: (i, 0))

    def full(a):
        return pl.BlockSpec(a.shape, lambda i: (0,) * a.ndim)

    return pl.pallas_call(
        body, grid=(g,),
        in_specs=[bs, bs, bs, full(W1), full(b1), full(W2), full(bs1)],
        out_specs=pl.BlockSpec((256, H), lambda i: (i, 0)),
        out_shape=jax.ShapeDtypeStruct((NP1, H), F32),
    )(tvals, svals, dis2, W1, b1, W2, bs1)


def _tc3(wacc, hh, dis2, b2, Ws2):
    def body(wa_r, wb_r, hh_r, d2_r, b2_r, ws2_r, h2_r, ss2_r):
        accraw = wa_r[...] + wb_r[...]
        agg2 = d2_r[...] * (accraw + hh_r[...]) + b2_r[...]
        h2 = jax.nn.relu(agg2)
        s2 = jnp.dot(h2, ws2_r[...], preferred_element_type=F32,
                     precision=HIGHEST)
        h2_r[...] = h2
        ss2_r[...] = d2_r[...] * s2

    g = NP1 // 256
    bw = pl.BlockSpec((256, H), lambda i: (i, 0))
    bwb = pl.BlockSpec((256, H), lambda i: (i + NP1 // 256, 0))
    bs = pl.BlockSpec((256, 1), lambda i: (i, 0))

    def full(a):
        return pl.BlockSpec(a.shape, lambda i: (0,) * a.ndim)

    return pl.pallas_call(
        body, grid=(g,),
        in_specs=[bw, bwb, bw, bs, full(b2), full(Ws2)],
        out_specs=[bw, bs],
        out_shape=[jax.ShapeDtypeStruct((NP1, H), F32),
                   jax.ShapeDtypeStruct((NP1, 1), F32)],
    )(wacc, wacc, hh, dis2, b2, Ws2)


def _tc4(h2sel, svals2, dis3, W3, bs2):
    def body(hs_r, sv_r, d3_r, w3_r, bs2_r, hh_r):
        sv = sv_r[...] + bs2_r[0, 0]
        xk2 = hs_r[...] * jnp.tanh(sv)
        h3pre = jnp.dot(xk2, w3_r[...], preferred_element_type=F32,
                        precision=HIGHEST)
        hh_r[...] = d3_r[...] * h3pre

    g = NP2 // 256
    bw = pl.BlockSpec((256, H), lambda i: (i, 0))
    bs = pl.BlockSpec((256, 1), lambda i: (i, 0))

    def full(a):
        return pl.BlockSpec(a.shape, lambda i: (0,) * a.ndim)

    return pl.pallas_call(
        body, grid=(g,),
        in_specs=[bw, bs, bs, full(W3), full(bs2)],
        out_specs=bw,
        out_shape=jax.ShapeDtypeStruct((NP2, H), F32),
    )(h2sel, svals2, dis3, W3, bs2)


def _tc_lstm(x2t, wih_t, whh_t, bias, wl_t, bl):
    T = x2t.shape[0]
    BP = x2t.shape[1]

    def body(x_r, wih_r, whh_r, b_r, wl_r, bl_r, out_r):
        whh = whh_r[...]
        wih = wih_r[...]
        bb = b_r[...]

        def step(t, hc):
            h, c = hc
            xt = x_r[t]
            gates = (jnp.dot(xt, wih, preferred_element_type=F32,
                             precision=HIGHEST)
                     + jnp.dot(h, whh, preferred_element_type=F32,
                               precision=HIGHEST) + bb)
            i = jax.nn.sigmoid(gates[:, 0:H])
            f = jax.nn.sigmoid(gates[:, H:2 * H])
            g = jnp.tanh(gates[:, 2 * H:3 * H])
            o = jax.nn.sigmoid(gates[:, 3 * H:4 * H])
            c = f * c + i * g
            h = o * jnp.tanh(c)
            return (h, c)

        h0 = jnp.zeros((BP, H), F32)
        h, _ = lax.fori_loop(0, T, step, (h0, h0))
        out_r[...] = jnp.dot(h, wl_r[...], preferred_element_type=F32,
                             precision=HIGHEST) + bl_r[...]

    def full(a):
        return pl.BlockSpec(a.shape, lambda: (0,) * a.ndim)

    return pl.pallas_call(
        body,
        in_specs=[full(x2t), full(wih_t), full(whh_t), full(bias),
                  full(wl_t), full(bl)],
        out_specs=pl.BlockSpec((BP, H), lambda: (0, 0)),
        out_shape=jax.ShapeDtypeStruct((BP, H), F32),
    )(x2t, wih_t, whh_t, bias, wl_t, bl)


def _tc5(wacc3, hh3, dis3, b3, S, x2o, wf_t, bf):
    def body(wa_r, hh_r, d3_r, b3_r, s_r, x2o_r, wf_r, bf_r, out_r):
        accraw = wa_r[0] + wa_r[1]
        h3 = d3_r[...] * (accraw + hh_r[...]) + b3_r[...]
        x1 = jnp.dot(s_r[...], h3, preferred_element_type=F32,
                     precision=HIGHEST)
        xc = jnp.concatenate([x1, x2o_r[...]], axis=1)
        out_r[...] = jnp.dot(xc, wf_r[...], preferred_element_type=F32,
                             precision=HIGHEST) + bf_r[...]

    def full(a):
        return pl.BlockSpec(a.shape, lambda: (0,) * a.ndim)

    bw = pl.BlockSpec((256, H), lambda: (0, 0))
    bwb = pl.BlockSpec((256, H), lambda: (NP1 // 256, 0))
    del bw, bwb
    return pl.pallas_call(
        body,
        in_specs=[pl.BlockSpec((2, NP2, H), lambda: (0, 0, 0)),
                  full(hh3),
                  pl.BlockSpec((NP2, 1), lambda: (0, 0)),
                  full(b3), full(S), full(x2o), full(wf_t), full(bf)],
        out_specs=pl.BlockSpec((64, 128), lambda: (0, 0)),
        out_shape=jax.ShapeDtypeStruct((64, 128), F32),
    )(wacc3, hh3, dis3, b3, S, x2o, wf_t, bf)


def kernel(x, edge_index, batch, x2, W1, b1, Ws1, bs1, W2, b2, Ws2, bs2,
           W3, b3, Wih, Whh, bih, bhh, Wl, bl, Wf, bf):
    del batch
    xp = jnp.zeros((NP,), F32).at[:N].set(x[:, 0])
    src = edge_index[0]
    dst = edge_index[1]
    padi = jnp.full((EROWS * 128 - E,), EPAD_IDX, I32)
    src1d = jnp.concatenate([src, padi])
    dst1d = jnp.concatenate([dst, padi])
    zw2 = jnp.zeros((NP1 * 16,), F32)
    zw3 = jnp.zeros((NP2 * 16,), F32)
    b1r = b1.reshape(1, H)
    b2r = b2.reshape(1, H)
    b3r = b3.reshape(1, H)
    bs1r = bs1.reshape(1, 1)
    bs2r = bs2.reshape(1, 1)

    dis, tacc = _sc1(xp, src1d, dst1d)
    t2d, ss12d = _tc1(dis.reshape(NP, 1), tacc.reshape(NP, 1),
                      xp.reshape(NP, 1), W1, b1r, Ws1)
    t = t2d.reshape(NP)
    ss1 = ss12d.reshape(NP)

    score = _scs1(ss1, dis, src1d, dst1d)
    map1 = _tc_rank(score[:N].reshape(B, N1G), N1G, K1, N1)
    map1 = jnp.concatenate([map1.reshape(N),
                            jnp.full((NP - N,), N1, I32)])

    nsc, ndc, rcnt, dis2, tvals, svals = _scp1(
        map1, src1d, dst1d, t, score)

    hh2 = _tc2(tvals.reshape(NP1, 1), svals.reshape(NP1, 1),
               dis2.reshape(NP1, 1), W1, b1r, W2, bs1r)
    wacc2 = _scw(hh2.reshape(NP1 * 2, 128), nsc, ndc, rcnt, zw2, NP1)
    wacc2 = (wacc2.reshape(2, 16, NP1, 16).transpose(0, 2, 1, 3)
             .reshape(2 * NP1, H))
    h2, ss2 = _tc3(wacc2, hh2, dis2.reshape(NP1, 1), b2r, Ws2)

    score2 = _scs2(ss2.reshape(NP1), dis2, nsc, ndc, rcnt)
    map2 = _tc_rank(score2[:N1].reshape(B, K1), K1, K2, N2)
    map2 = jnp.concatenate([map2.reshape(N1),
                            jnp.full((NP1 - N1,), N2, I32)])

    nsc2, ndc2, rcnt2, dis3, h2sel, svals2 = _scp2(
        map2, nsc, ndc, rcnt, h2, score2)

    hh3 = _tc4(h2sel, svals2.reshape(NP2, 1), dis3.reshape(NP2, 1),
               W3, bs2r)
    wacc3 = _scw(hh3.reshape(NP2 * 2, 128), nsc2, ndc2, rcnt2, zw3, NP2)
    wacc3 = (wacc3.reshape(2, 16, NP2, 16).transpose(0, 2, 1, 3)
             .reshape(2, NP2, H))

    x2t = jnp.zeros((x2.shape[1], 64, 8), F32).at[:, :B, :6].set(
        jnp.swapaxes(x2, 0, 1))
    wih_t = jnp.zeros((8, 4 * H), F32).at[:6, :].set(Wih.T)
    bias = (bih + bhh).reshape(1, 4 * H)
    x2o = _tc_lstm(x2t, wih_t, Whh.T, bias, Wl.T, bl.reshape(1, H))

    S = jnp.zeros((64, NP2), F32).at[
        jnp.repeat(jnp.arange(B), K2), jnp.arange(N2)].set(1.0 / K2)
    wf_t = jnp.zeros((2 * H, 128), F32).at[:, :2].set(Wf.T)
    bfp = jnp.zeros((1, 128), F32).at[0, :2].set(bf)
    out = _tc5(wacc3, hh3, dis3.reshape(NP2, 1), b3r, S, x2o, wf_t, bfp)
    return out[:B, :2]
